# Initial kernel scaffold; baseline (speedup 1.0000x reference)
import jax
import jax.numpy as jnp
from jax.experimental import pallas as pl


def kernel(x, edge_index, W1, a_src1, a_dst1, b1, W2, a_src2, a_dst2, b2):
    raise NotImplementedError("write your pallas kernel here")


# speedup vs baseline: 26.4766x; 26.4766x over previous
import functools

import jax
import jax.numpy as jnp
from jax import lax
from jax.experimental import pallas as pl
from jax.experimental.pallas import tpu as pltpu
from jax.experimental.pallas import tpu_sc as plsc

N = 10000
E = 320000
D_IN = 128
H1, C1 = 8, 16
H2, C2 = 1, 64
HH = H1 // 2
CH = C2 // 2
R1 = 72
A1 = 80
R2 = 32
A2 = 48

NC, NS = 2, 16
EPT = E // NS
CSZ = 80
NCH = EPT // CSZ
NPH = 10112
ZR = NPH // NS

_mesh = plsc.VectorSubcoreMesh(core_axis_name="c", subcore_axis_name="s")
_sc_params = pltpu.CompilerParams(use_tc_tiling_on_sc=False,
                                  needs_layout_passes=False)


def _zero_fill(buf, rows, width):
    def _zrow(r, carry):
        for k in range(width // 16):
            buf[r, pl.ds(k * 16, 16)] = jnp.zeros((16,), jnp.float32)
        return carry
    lax.fori_loop(0, rows, _zrow, 0)


def _zero_acc_rows(acc, zbuf, zbase):
    for t in range(ZR // CSZ):
        pltpu.sync_copy(zbuf, acc.at[pl.ds(zbase + t * CSZ, CSZ)])
    rem = ZR - (ZR // CSZ) * CSZ
    if rem:
        pltpu.sync_copy(zbuf.at[pl.ds(0, rem)],
                        acc.at[pl.ds(zbase + ZR - rem, rem)])


@functools.partial(
    pl.kernel,
    out_type=jax.ShapeDtypeStruct((NC, NPH, A1), jnp.float32),
    mesh=_mesh,
    compiler_params=_sc_params,
    scratch_types=[
        pltpu.VMEM((CSZ, R1), jnp.float32),
        pltpu.VMEM((CSZ, A1), jnp.float32),
        pltpu.VMEM((CSZ, 8), jnp.float32),
        pltpu.VMEM((CSZ,), jnp.int32),
        pltpu.VMEM((CSZ,), jnp.int32),
        pltpu.VMEM((CSZ,), jnp.int32),
        pltpu.VMEM_SHARED((NPH, A1), jnp.float32),
        pltpu.SemaphoreType.DMA,
        pltpu.SemaphoreType.DMA,
    ],
)
def _l1_edges(tab_hbm, adst_hbm, esrc_hbm, edst_hbm, outm_hbm,
              buf_in, buf_out, dbuf, sidx, didx, didx2, acc_sh, gsem, dsem):
    cid = lax.axis_index("c")
    sid = lax.axis_index("s")
    ebase = sid * EPT
    rowbias = cid * N

    _zero_fill(buf_out, CSZ, A1)
    _zero_acc_rows(acc_sh, buf_out, sid * ZR)
    plsc.subcore_barrier()

    iota = lax.iota(jnp.int32, 16)

    def _chunk(j, carry):
        eb = ebase + j * CSZ
        pltpu.sync_copy(esrc_hbm.at[pl.ds(eb, CSZ)], sidx)
        pltpu.sync_copy(edst_hbm.at[pl.ds(eb, CSZ)], didx)
        for g in range(CSZ // 16):
            sidx[pl.ds(g * 16, 16)] = sidx[pl.ds(g * 16, 16)] + rowbias
            didx2[pl.ds(g * 16, 16)] = didx[pl.ds(g * 16, 16)] + rowbias
        cpa = pltpu.async_copy(tab_hbm.at[sidx], buf_in, gsem)
        cpb = pltpu.async_copy(adst_hbm.at[didx2], dbuf, dsem)
        cpa.wait()
        cpb.wait()
        for g in range(CSZ // 16):
            e16 = iota + g * 16
            for hd in range(HH):
                hdv = jnp.full((16,), hd, jnp.int32)
                a_s = plsc.load_gather(buf_in, [e16, hdv])
                a_d = plsc.load_gather(dbuf, [e16, hdv])
                ew = a_s + a_d
                ew = jnp.where(ew > 0.0, ew, 0.2 * ew)
                w = jnp.exp(ew)
                plsc.store_scatter(buf_out, [e16, hdv], w)

        def _edge(i, carry2):
            wv = buf_out[i, pl.ds(0, 16)]
            for hd in range(HH):
                ws = wv[hd]
                buf_out[i, pl.ds(16 + hd * 16, 16)] = (
                    buf_in[i, pl.ds(8 + hd * 16, 16)] * ws)
            return carry2
        lax.fori_loop(0, CSZ, _edge, 0)
        pltpu.sync_copy(buf_out, acc_sh.at[didx], add=True)
        return carry
    lax.fori_loop(0, NCH, _chunk, 0)

    plsc.subcore_barrier()
    pltpu.sync_copy(acc_sh.at[pl.ds(sid * ZR, ZR)],
                    outm_hbm.at[cid, pl.ds(sid * ZR, ZR)])


@functools.partial(
    pl.kernel,
    out_type=jax.ShapeDtypeStruct((NC, NPH, A2), jnp.float32),
    mesh=_mesh,
    compiler_params=_sc_params,
    scratch_types=[
        pltpu.VMEM((CSZ, R2), jnp.float32),
        pltpu.VMEM((CSZ, A2), jnp.float32),
        pltpu.VMEM((CSZ, 8), jnp.float32),
        pltpu.VMEM((CSZ, 8), jnp.float32),
        pltpu.VMEM((CSZ,), jnp.int32),
        pltpu.VMEM((CSZ,), jnp.int32),
        pltpu.VMEM((CSZ,), jnp.int32),
        pltpu.VMEM_SHARED((NPH, A2), jnp.float32),
        pltpu.SemaphoreType.DMA,
        pltpu.SemaphoreType.DMA,
        pltpu.SemaphoreType.DMA,
    ],
)
def _l2_edges(tab_hbm, a2_hbm, esrc_hbm, edst_hbm, outm_hbm,
              buf_in, buf_out, abuf_s, abuf_d, sidx, sidx2, didx, accm_sh,
              gsem, ssem, dsem):
    cid = lax.axis_index("c")
    sid = lax.axis_index("s")
    ebase = sid * EPT
    rowbias = cid * N

    _zero_fill(buf_out, CSZ, A2)
    _zero_acc_rows(accm_sh, buf_out, sid * ZR)
    plsc.subcore_barrier()

    iota = lax.iota(jnp.int32, 16)
    zero16 = jnp.zeros((16,), jnp.int32)
    one16 = jnp.ones((16,), jnp.int32)

    def _chunk(j, carry):
        eb = ebase + j * CSZ
        pltpu.sync_copy(esrc_hbm.at[pl.ds(eb, CSZ)], sidx)
        pltpu.sync_copy(edst_hbm.at[pl.ds(eb, CSZ)], didx)
        for g in range(CSZ // 16):
            sidx2[pl.ds(g * 16, 16)] = sidx[pl.ds(g * 16, 16)] + rowbias
        cpa = pltpu.async_copy(tab_hbm.at[sidx2], buf_in, gsem)
        cpb = pltpu.async_copy(a2_hbm.at[sidx], abuf_s, ssem)
        cpc = pltpu.async_copy(a2_hbm.at[didx], abuf_d, dsem)
        cpa.wait()
        cpb.wait()
        cpc.wait()
        for g in range(CSZ // 16):
            e16 = iota + g * 16
            a_s = plsc.load_gather(abuf_s, [e16, zero16])
            a_d = plsc.load_gather(abuf_d, [e16, one16])
            ew = a_s + a_d
            ew = jnp.where(ew > 0.0, ew, 0.2 * ew)
            w = jnp.exp(ew)
            plsc.store_scatter(buf_out, [e16, zero16], w)

        def _edge(i, carry2):
            ws = buf_out[i, pl.ds(0, 16)][0]
            for k in range(R2 // 16):
                buf_out[i, pl.ds(16 + k * 16, 16)] = (
                    buf_in[i, pl.ds(k * 16, 16)] * ws)
            return carry2
        lax.fori_loop(0, CSZ, _edge, 0)
        pltpu.sync_copy(buf_out, accm_sh.at[didx], add=True)
        return carry
    lax.fori_loop(0, NCH, _chunk, 0)

    plsc.subcore_barrier()
    pltpu.sync_copy(accm_sh.at[pl.ds(sid * ZR, ZR)],
                    outm_hbm.at[cid, pl.ds(sid * ZR, ZR)])


BR = 400
NB = N // BR


def _tc_pre_body(x_ref, w_ref, as_ref, ad_ref,
                 taba_ref, tabb_ref, adsta_ref, adstb_ref):
    h = jnp.dot(x_ref[...], w_ref[...], preferred_element_type=jnp.float32)
    hr = h.reshape(BR, H1, C1)
    asrc = (hr * as_ref[...][None]).sum(-1)
    adst = (hr * ad_ref[...][None]).sum(-1)
    z = jnp.zeros((BR, 4), jnp.float32)
    taba_ref[...] = jnp.concatenate(
        [asrc[:, 0:HH], z, h[:, 0:HH * C1]], axis=1)
    tabb_ref[...] = jnp.concatenate(
        [asrc[:, HH:H1], z, h[:, HH * C1:D_IN]], axis=1)
    adsta_ref[...] = jnp.concatenate([adst[:, 0:HH], z], axis=1)
    adstb_ref[...] = jnp.concatenate([adst[:, HH:H1], z], axis=1)


def _tc_mid_body(accm_ref, b1_ref, w2_ref, as2_ref, ad2_ref,
                 tab2a_ref, tab2b_ref, a2tab_ref):
    wsum = jnp.concatenate(
        [accm_ref[0, :, 0:HH], accm_ref[1, :, 0:HH]], axis=1)
    msg = jnp.concatenate(
        [accm_ref[0, :, 16:A1], accm_ref[1, :, 16:A1]], axis=1)
    o = msg.reshape(BR, H1, C1) / (wsum[:, :, None] + 1e-16)
    o = (o + b1_ref[...].reshape(1, H1, C1)).reshape(BR, D_IN)
    o = jnp.where(o > 0.0, o, jnp.exp(o) - 1.0)
    h2 = jnp.dot(o, w2_ref[...], preferred_element_type=jnp.float32)
    tab2a_ref[...] = h2[:, 0:CH]
    tab2b_ref[...] = h2[:, CH:C2]
    asrc2 = (h2 * as2_ref[...]).sum(-1, keepdims=True)
    adst2 = (h2 * ad2_ref[...]).sum(-1, keepdims=True)
    a2tab_ref[...] = jnp.concatenate(
        [asrc2, adst2, jnp.zeros((BR, 6), jnp.float32)], axis=1)


def _tc_post_body(accm_ref, b2_ref, out_ref):
    msg = jnp.concatenate(
        [accm_ref[0, :, 16:A2], accm_ref[1, :, 16:A2]], axis=1)
    w = accm_ref[0, :, 0:1]
    out_ref[...] = msg / (w + 1e-16) + b2_ref[...]


_pre_call = pl.pallas_call(
    _tc_pre_body,
    grid=(NB,),
    in_specs=[
        pl.BlockSpec((BR, D_IN), lambda i: (i, 0)),
        pl.BlockSpec((D_IN, H1 * C1), lambda i: (0, 0)),
        pl.BlockSpec((H1, C1), lambda i: (0, 0)),
        pl.BlockSpec((H1, C1), lambda i: (0, 0)),
    ],
    out_specs=[
        pl.BlockSpec((BR, R1), lambda i: (i, 0)),
        pl.BlockSpec((BR, R1), lambda i: (i, 0)),
        pl.BlockSpec((BR, 8), lambda i: (i, 0)),
        pl.BlockSpec((BR, 8), lambda i: (i, 0)),
    ],
    out_shape=[
        jax.ShapeDtypeStruct((N, R1), jnp.float32),
        jax.ShapeDtypeStruct((N, R1), jnp.float32),
        jax.ShapeDtypeStruct((N, 8), jnp.float32),
        jax.ShapeDtypeStruct((N, 8), jnp.float32),
    ],
)

_mid_call = pl.pallas_call(
    _tc_mid_body,
    grid=(NB,),
    in_specs=[
        pl.BlockSpec((NC, BR, A1), lambda i: (0, i, 0)),
        pl.BlockSpec((1, D_IN), lambda i: (0, 0)),
        pl.BlockSpec((D_IN, H2 * C2), lambda i: (0, 0)),
        pl.BlockSpec((1, C2), lambda i: (0, 0)),
        pl.BlockSpec((1, C2), lambda i: (0, 0)),
    ],
    out_specs=[
        pl.BlockSpec((BR, CH), lambda i: (i, 0)),
        pl.BlockSpec((BR, CH), lambda i: (i, 0)),
        pl.BlockSpec((BR, 8), lambda i: (i, 0)),
    ],
    out_shape=[
        jax.ShapeDtypeStruct((N, CH), jnp.float32),
        jax.ShapeDtypeStruct((N, CH), jnp.float32),
        jax.ShapeDtypeStruct((N, 8), jnp.float32),
    ],
)

_post_call = pl.pallas_call(
    _tc_post_body,
    grid=(NB,),
    in_specs=[
        pl.BlockSpec((NC, BR, A2), lambda i: (0, i, 0)),
        pl.BlockSpec((1, C2), lambda i: (0, 0)),
    ],
    out_specs=pl.BlockSpec((BR, C2), lambda i: (i, 0)),
    out_shape=jax.ShapeDtypeStruct((N, C2), jnp.float32),
)


def kernel(x, edge_index, W1, a_src1, a_dst1, b1, W2, a_src2, a_dst2, b2):
    esrc, edst = edge_index[0], edge_index[1]
    taba, tabb, adsta, adstb = _pre_call(x, W1, a_src1, a_dst1)
    tab1 = jnp.concatenate([taba, tabb], axis=0)
    adst1 = jnp.concatenate([adsta, adstb], axis=0)
    accm1 = _l1_edges(tab1, adst1, esrc, edst)
    accm1 = accm1[:, 0:N]
    tab2a, tab2b, a2tab = _mid_call(accm1, b1.reshape(1, D_IN),
                                    W2, a_src2.reshape(1, C2),
                                    a_dst2.reshape(1, C2))
    tab2 = jnp.concatenate([tab2a, tab2b], axis=0)
    accm2 = _l2_edges(tab2, a2tab, esrc, edst)
    accm2 = accm2[:, 0:N]
    return _post_call(accm2, b2.reshape(1, C2))


# speedup vs baseline: 41.3781x; 1.5628x over previous
import functools

import jax
import jax.numpy as jnp
from jax import lax
from jax.experimental import pallas as pl
from jax.experimental.pallas import tpu as pltpu
from jax.experimental.pallas import tpu_sc as plsc

N = 10000
E = 320000
D_IN = 128
H1, C1 = 8, 16
H2, C2 = 1, 64
HH = H1 // 2
CH = C2 // 2
R1 = 72
A1 = 80
R2 = 40
A2 = 48

NC, NS = 2, 16
EPT = E // NS
CSZ = 80
NCH = EPT // CSZ
SB = 10
NSB = NCH // SB
DEPTH = 4
NPH = 10112
ZR = NPH // NS

_mesh = plsc.VectorSubcoreMesh(core_axis_name="c", subcore_axis_name="s")
_sc_params = pltpu.CompilerParams(use_tc_tiling_on_sc=False,
                                  needs_layout_passes=False)


def _zero_fill(buf, rows, width):
    def _zrow(r, carry):
        for k in range(width // 16):
            buf[r, pl.ds(k * 16, 16)] = jnp.zeros((16,), jnp.float32)
        return carry
    lax.fori_loop(0, rows, _zrow, 0)


def _zero_acc_rows(acc, zbuf, zbase):
    for t in range(ZR // CSZ):
        pltpu.sync_copy(zbuf, acc.at[pl.ds(zbase + t * CSZ, CSZ)])
    rem = ZR - (ZR // CSZ) * CSZ
    if rem:
        pltpu.sync_copy(zbuf.at[pl.ds(0, rem)],
                        acc.at[pl.ds(zbase + ZR - rem, rem)])


@functools.partial(
    pl.kernel,
    out_type=jax.ShapeDtypeStruct((NC, NPH, A1), jnp.float32),
    mesh=_mesh,
    compiler_params=_sc_params,
    scratch_types=(
        [pltpu.VMEM((CSZ, R1), jnp.float32)] * DEPTH
        + [pltpu.VMEM((CSZ, 8), jnp.float32)] * DEPTH
        + [
            pltpu.VMEM((CSZ, A1), jnp.float32),
            pltpu.VMEM((SB * CSZ,), jnp.int32),
            pltpu.VMEM((SB * CSZ,), jnp.int32),
            pltpu.VMEM((SB * CSZ,), jnp.int32),
            pltpu.VMEM_SHARED((NPH, A1), jnp.float32),
        ]
        + [pltpu.SemaphoreType.DMA] * (2 * DEPTH)
    ),
)
def _l1_edges(tab_hbm, adst_hbm, esrc_hbm, edst_hbm, outm_hbm, *refs):
    bufi = refs[0:DEPTH]
    dbuf = refs[DEPTH:2 * DEPTH]
    buf_out, sidxb, didxb, didx2b, acc_sh = refs[2 * DEPTH:2 * DEPTH + 5]
    gsem = refs[2 * DEPTH + 5:2 * DEPTH + 5 + DEPTH]
    dsem = refs[2 * DEPTH + 5 + DEPTH:]
    cid = lax.axis_index("c")
    sid = lax.axis_index("s")
    ebase = sid * EPT
    rowbias = cid * N

    _zero_fill(buf_out, CSZ, A1)
    _zero_acc_rows(acc_sh, buf_out, sid * ZR)
    plsc.subcore_barrier()

    iota = lax.iota(jnp.int32, 16)

    def _super(s, carry):
        eb = ebase + s * (SB * CSZ)
        pltpu.sync_copy(esrc_hbm.at[pl.ds(eb, SB * CSZ)], sidxb)
        pltpu.sync_copy(edst_hbm.at[pl.ds(eb, SB * CSZ)], didxb)

        def _bias(g, c2):
            sl = pl.ds(g * 16, 16)
            sidxb[sl] = sidxb[sl] + rowbias
            didx2b[sl] = didxb[sl] + rowbias
            return c2
        lax.fori_loop(0, SB * CSZ // 16, _bias, 0)

        def _issue(k):
            sl = pl.ds(k * CSZ, CSZ)
            b = k % DEPTH
            return (
                pltpu.async_copy(tab_hbm.at[sidxb.at[sl]], bufi[b], gsem[b]),
                pltpu.async_copy(adst_hbm.at[didx2b.at[sl]], dbuf[b], dsem[b]),
            )

        hs = {k: _issue(k) for k in range(DEPTH)}
        for k in range(SB):
            ha, hb = hs.pop(k)
            ha.wait()
            hb.wait()
            b = k % DEPTH
            for g in range(CSZ // 16):
                e16 = iota + g * 16
                for hd in range(HH):
                    hdv = jnp.full((16,), hd, jnp.int32)
                    a_s = plsc.load_gather(bufi[b], [e16, hdv])
                    a_d = plsc.load_gather(dbuf[b], [e16, hdv])
                    ew = a_s + a_d
                    ew = jnp.where(ew > 0.0, ew, 0.2 * ew)
                    w = jnp.exp(ew)
                    plsc.store_scatter(buf_out, [e16, hdv], w)

            def _edge(i, carry2, b=b):
                wv = buf_out[i, pl.ds(0, 16)]
                for hd in range(HH):
                    ws = wv[hd]
                    buf_out[i, pl.ds(16 + hd * 16, 16)] = (
                        bufi[b][i, pl.ds(8 + hd * 16, 16)] * ws)
                return carry2
            lax.fori_loop(0, CSZ, _edge, 0)
            pltpu.sync_copy(buf_out,
                            acc_sh.at[didxb.at[pl.ds(k * CSZ, CSZ)]],
                            add=True)
            if k + DEPTH < SB:
                hs[k + DEPTH] = _issue(k + DEPTH)
        return carry
    lax.fori_loop(0, NSB, _super, 0)

    plsc.subcore_barrier()
    pltpu.sync_copy(acc_sh.at[pl.ds(sid * ZR, ZR)],
                    outm_hbm.at[cid, pl.ds(sid * ZR, ZR)])


@functools.partial(
    pl.kernel,
    out_type=jax.ShapeDtypeStruct((NC, NPH, A2), jnp.float32),
    mesh=_mesh,
    compiler_params=_sc_params,
    scratch_types=(
        [pltpu.VMEM((CSZ, R2), jnp.float32)] * DEPTH
        + [pltpu.VMEM((CSZ, 8), jnp.float32)] * DEPTH
        + [
            pltpu.VMEM((CSZ, A2), jnp.float32),
            pltpu.VMEM((SB * CSZ,), jnp.int32),
            pltpu.VMEM((SB * CSZ,), jnp.int32),
            pltpu.VMEM_SHARED((NPH, A2), jnp.float32),
        ]
        + [pltpu.SemaphoreType.DMA] * (2 * DEPTH)
    ),
)
def _l2_edges(tab_hbm, a2_hbm, esrc_hbm, edst_hbm, outm_hbm, *refs):
    bufi = refs[0:DEPTH]
    dbuf = refs[DEPTH:2 * DEPTH]
    buf_out, sidxb, didxb, accm_sh = refs[2 * DEPTH:2 * DEPTH + 4]
    gsem = refs[2 * DEPTH + 4:2 * DEPTH + 4 + DEPTH]
    dsem = refs[2 * DEPTH + 4 + DEPTH:]
    cid = lax.axis_index("c")
    sid = lax.axis_index("s")
    ebase = sid * EPT
    rowbias = cid * N

    _zero_fill(buf_out, CSZ, A2)
    _zero_acc_rows(accm_sh, buf_out, sid * ZR)
    plsc.subcore_barrier()

    iota = lax.iota(jnp.int32, 16)
    zero16 = jnp.zeros((16,), jnp.int32)

    def _super(s, carry):
        eb = ebase + s * (SB * CSZ)
        pltpu.sync_copy(esrc_hbm.at[pl.ds(eb, SB * CSZ)], sidxb)
        pltpu.sync_copy(edst_hbm.at[pl.ds(eb, SB * CSZ)], didxb)

        def _bias(g, c2):
            sl = pl.ds(g * 16, 16)
            sidxb[sl] = sidxb[sl] + rowbias
            return c2
        lax.fori_loop(0, SB * CSZ // 16, _bias, 0)

        def _issue(k):
            sl = pl.ds(k * CSZ, CSZ)
            b = k % DEPTH
            return (
                pltpu.async_copy(tab_hbm.at[sidxb.at[sl]], bufi[b], gsem[b]),
                pltpu.async_copy(a2_hbm.at[didxb.at[sl]], dbuf[b], dsem[b]),
            )

        hs = {k: _issue(k) for k in range(DEPTH)}
        for k in range(SB):
            ha, hb = hs.pop(k)
            ha.wait()
            hb.wait()
            b = k % DEPTH
            for g in range(CSZ // 16):
                e16 = iota + g * 16
                a_s = plsc.load_gather(bufi[b], [e16, zero16])
                a_d = plsc.load_gather(dbuf[b], [e16, zero16])
                ew = a_s + a_d
                ew = jnp.where(ew > 0.0, ew, 0.2 * ew)
                w = jnp.exp(ew)
                plsc.store_scatter(buf_out, [e16, zero16], w)

            def _edge(i, carry2, b=b):
                ws = buf_out[i, pl.ds(0, 16)][0]
                for kk in range((R2 - 8) // 16):
                    buf_out[i, pl.ds(16 + kk * 16, 16)] = (
                        bufi[b][i, pl.ds(8 + kk * 16, 16)] * ws)
                return carry2
            lax.fori_loop(0, CSZ, _edge, 0)
            pltpu.sync_copy(buf_out,
                            accm_sh.at[didxb.at[pl.ds(k * CSZ, CSZ)]],
                            add=True)
            if k + DEPTH < SB:
                hs[k + DEPTH] = _issue(k + DEPTH)
        return carry
    lax.fori_loop(0, NSB, _super, 0)

    plsc.subcore_barrier()
    pltpu.sync_copy(accm_sh.at[pl.ds(sid * ZR, ZR)],
                    outm_hbm.at[cid, pl.ds(sid * ZR, ZR)])


BR = 400
NB = N // BR


def _tc_pre_body(x_ref, w_ref, as_ref, ad_ref,
                 taba_ref, tabb_ref, adsta_ref, adstb_ref):
    h = jnp.dot(x_ref[...], w_ref[...], preferred_element_type=jnp.float32)
    hr = h.reshape(BR, H1, C1)
    asrc = (hr * as_ref[...][None]).sum(-1)
    adst = (hr * ad_ref[...][None]).sum(-1)
    z = jnp.zeros((BR, 4), jnp.float32)
    taba_ref[...] = jnp.concatenate(
        [asrc[:, 0:HH], z, h[:, 0:HH * C1]], axis=1)
    tabb_ref[...] = jnp.concatenate(
        [asrc[:, HH:H1], z, h[:, HH * C1:D_IN]], axis=1)
    adsta_ref[...] = jnp.concatenate([adst[:, 0:HH], z], axis=1)
    adstb_ref[...] = jnp.concatenate([adst[:, HH:H1], z], axis=1)


def _tc_mid_body(accm_ref, b1_ref, w2_ref, as2_ref, ad2_ref,
                 tab2a_ref, tab2b_ref, a2tab_ref):
    wsum = jnp.concatenate(
        [accm_ref[0, :, 0:HH], accm_ref[1, :, 0:HH]], axis=1)
    msg = jnp.concatenate(
        [accm_ref[0, :, 16:A1], accm_ref[1, :, 16:A1]], axis=1)
    o = msg.reshape(BR, H1, C1) / (wsum[:, :, None] + 1e-16)
    o = (o + b1_ref[...].reshape(1, H1, C1)).reshape(BR, D_IN)
    o = jnp.where(o > 0.0, o, jnp.exp(o) - 1.0)
    h2 = jnp.dot(o, w2_ref[...], preferred_element_type=jnp.float32)
    asrc2 = (h2 * as2_ref[...]).sum(-1, keepdims=True)
    adst2 = (h2 * ad2_ref[...]).sum(-1, keepdims=True)
    z7 = jnp.zeros((BR, 7), jnp.float32)
    tab2a_ref[...] = jnp.concatenate([asrc2, z7, h2[:, 0:CH]], axis=1)
    tab2b_ref[...] = jnp.concatenate([asrc2, z7, h2[:, CH:C2]], axis=1)
    a2tab_ref[...] = jnp.concatenate([adst2, z7], axis=1)


def _tc_post_body(accm_ref, b2_ref, out_ref):
    msg = jnp.concatenate(
        [accm_ref[0, :, 16:A2], accm_ref[1, :, 16:A2]], axis=1)
    w = accm_ref[0, :, 0:1]
    out_ref[...] = msg / (w + 1e-16) + b2_ref[...]


_pre_call = pl.pallas_call(
    _tc_pre_body,
    grid=(NB,),
    in_specs=[
        pl.BlockSpec((BR, D_IN), lambda i: (i, 0)),
        pl.BlockSpec((D_IN, H1 * C1), lambda i: (0, 0)),
        pl.BlockSpec((H1, C1), lambda i: (0, 0)),
        pl.BlockSpec((H1, C1), lambda i: (0, 0)),
    ],
    out_specs=[
        pl.BlockSpec((BR, R1), lambda i: (i, 0)),
        pl.BlockSpec((BR, R1), lambda i: (i, 0)),
        pl.BlockSpec((BR, 8), lambda i: (i, 0)),
        pl.BlockSpec((BR, 8), lambda i: (i, 0)),
    ],
    out_shape=[
        jax.ShapeDtypeStruct((N, R1), jnp.float32),
        jax.ShapeDtypeStruct((N, R1), jnp.float32),
        jax.ShapeDtypeStruct((N, 8), jnp.float32),
        jax.ShapeDtypeStruct((N, 8), jnp.float32),
    ],
)

_mid_call = pl.pallas_call(
    _tc_mid_body,
    grid=(NB,),
    in_specs=[
        pl.BlockSpec((NC, BR, A1), lambda i: (0, i, 0)),
        pl.BlockSpec((1, D_IN), lambda i: (0, 0)),
        pl.BlockSpec((D_IN, H2 * C2), lambda i: (0, 0)),
        pl.BlockSpec((1, C2), lambda i: (0, 0)),
        pl.BlockSpec((1, C2), lambda i: (0, 0)),
    ],
    out_specs=[
        pl.BlockSpec((BR, R2), lambda i: (i, 0)),
        pl.BlockSpec((BR, R2), lambda i: (i, 0)),
        pl.BlockSpec((BR, 8), lambda i: (i, 0)),
    ],
    out_shape=[
        jax.ShapeDtypeStruct((N, R2), jnp.float32),
        jax.ShapeDtypeStruct((N, R2), jnp.float32),
        jax.ShapeDtypeStruct((N, 8), jnp.float32),
    ],
)

_post_call = pl.pallas_call(
    _tc_post_body,
    grid=(NB,),
    in_specs=[
        pl.BlockSpec((NC, BR, A2), lambda i: (0, i, 0)),
        pl.BlockSpec((1, C2), lambda i: (0, 0)),
    ],
    out_specs=pl.BlockSpec((BR, C2), lambda i: (i, 0)),
    out_shape=jax.ShapeDtypeStruct((N, C2), jnp.float32),
)


def kernel(x, edge_index, W1, a_src1, a_dst1, b1, W2, a_src2, a_dst2, b2):
    esrc, edst = edge_index[0], edge_index[1]
    taba, tabb, adsta, adstb = _pre_call(x, W1, a_src1, a_dst1)
    tab1 = jnp.concatenate([taba, tabb], axis=0)
    adst1 = jnp.concatenate([adsta, adstb], axis=0)
    accm1 = _l1_edges(tab1, adst1, esrc, edst)
    accm1 = accm1[:, 0:N]
    tab2a, tab2b, a2tab = _mid_call(accm1, b1.reshape(1, D_IN),
                                    W2, a_src2.reshape(1, C2),
                                    a_dst2.reshape(1, C2))
    tab2 = jnp.concatenate([tab2a, tab2b], axis=0)
    accm2 = _l2_edges(tab2, a2tab, esrc, edst)
    accm2 = accm2[:, 0:N]
    return _post_call(accm2, b2.reshape(1, C2))


# speedup vs baseline: 60.8075x; 1.4696x over previous
import functools

import jax
import jax.numpy as jnp
from jax import lax
from jax.experimental import pallas as pl
from jax.experimental.pallas import tpu as pltpu
from jax.experimental.pallas import tpu_sc as plsc

N = 10000
E = 320000
D_IN = 128
H1, C1 = 8, 16
H2, C2 = 1, 64
HH = H1 // 2
CH = C2 // 2
R1 = 80
A1 = 80
R2 = 48
A2 = 48

NC, NS = 2, 16
EPT = E // NS
CSZ = 80
NCH = EPT // CSZ
SB = 10
NSB = NCH // SB
DEPTH = 4
NPH = 10112
ZR = NPH // NS

_mesh = plsc.VectorSubcoreMesh(core_axis_name="c", subcore_axis_name="s")
_sc_params = pltpu.CompilerParams(use_tc_tiling_on_sc=False,
                                  needs_layout_passes=False)


def _zero_fill(buf, rows, width):
    def _zrow(r, carry):
        for k in range(width // 16):
            buf[r, pl.ds(k * 16, 16)] = jnp.zeros((16,), jnp.float32)
        return carry
    lax.fori_loop(0, rows, _zrow, 0)


def _zero_acc_rows(acc, zbuf, zbase):
    for t in range(ZR // CSZ):
        pltpu.sync_copy(zbuf, acc.at[pl.ds(zbase + t * CSZ, CSZ)])
    rem = ZR - (ZR // CSZ) * CSZ
    if rem:
        pltpu.sync_copy(zbuf.at[pl.ds(0, rem)],
                        acc.at[pl.ds(zbase + ZR - rem, rem)])


@functools.partial(
    pl.kernel,
    out_type=jax.ShapeDtypeStruct((NC, NPH, A1), jnp.float32),
    mesh=_mesh,
    compiler_params=_sc_params,
    scratch_types=(
        [pltpu.VMEM((CSZ, R1), jnp.float32)] * DEPTH
        + [pltpu.VMEM((CSZ, 8), jnp.float32)] * DEPTH
        + [
            pltpu.VMEM((SB * CSZ,), jnp.int32),
            pltpu.VMEM((SB * CSZ,), jnp.int32),
            pltpu.VMEM((SB * CSZ,), jnp.int32),
            pltpu.VMEM_SHARED((NPH, A1), jnp.float32),
        ]
        + [pltpu.SemaphoreType.DMA] * (2 * DEPTH)
    ),
)
def _l1_edges(tab_hbm, adst_hbm, esrc_hbm, edst_hbm, outm_hbm, *refs):
    bufi = refs[0:DEPTH]
    dbuf = refs[DEPTH:2 * DEPTH]
    sidxb, didxb, didx2b, acc_sh = refs[2 * DEPTH:2 * DEPTH + 4]
    gsem = refs[2 * DEPTH + 4:2 * DEPTH + 4 + DEPTH]
    dsem = refs[2 * DEPTH + 4 + DEPTH:]
    cid = lax.axis_index("c")
    sid = lax.axis_index("s")
    ebase = sid * EPT
    rowbias = cid * N

    _zero_fill(bufi[0], CSZ, A1)
    _zero_acc_rows(acc_sh, bufi[0], sid * ZR)
    plsc.subcore_barrier()

    iota = lax.iota(jnp.int32, 16)

    def _super(s, carry):
        eb = ebase + s * (SB * CSZ)
        pltpu.sync_copy(esrc_hbm.at[pl.ds(eb, SB * CSZ)], sidxb)
        pltpu.sync_copy(edst_hbm.at[pl.ds(eb, SB * CSZ)], didxb)

        def _bias(g, c2):
            sl = pl.ds(g * 16, 16)
            sidxb[sl] = sidxb[sl] + rowbias
            didx2b[sl] = didxb[sl] + rowbias
            return c2
        lax.fori_loop(0, SB * CSZ // 16, _bias, 0)

        def _issue(k):
            sl = pl.ds(k * CSZ, CSZ)
            b = k % DEPTH
            return (
                pltpu.async_copy(tab_hbm.at[sidxb.at[sl]], bufi[b], gsem[b]),
                pltpu.async_copy(adst_hbm.at[didx2b.at[sl]], dbuf[b], dsem[b]),
            )

        hs = {k: _issue(k) for k in range(DEPTH)}
        for k in range(SB):
            ha, hb = hs.pop(k)
            ha.wait()
            hb.wait()
            b = k % DEPTH
            for g in range(CSZ // 16):
                e16 = iota + g * 16
                for hd in range(HH):
                    hdv = jnp.full((16,), hd, jnp.int32)
                    a_s = plsc.load_gather(bufi[b], [e16, hdv])
                    a_d = plsc.load_gather(dbuf[b], [e16, hdv])
                    ew = a_s + a_d
                    ew = jnp.where(ew > 0.0, ew, 0.2 * ew)
                    w = jnp.exp(ew)
                    plsc.store_scatter(bufi[b], [e16, hdv], w)

            def _edge(q, carry2, b=b):
                for u in range(4):
                    i = q * 4 + u
                    wv = bufi[b][i, pl.ds(0, 16)]
                    for hd in range(HH):
                        ws = wv[hd]
                        sl = pl.ds(16 + hd * 16, 16)
                        bufi[b][i, sl] = bufi[b][i, sl] * ws
                return carry2
            lax.fori_loop(0, CSZ // 4, _edge, 0)
            pltpu.sync_copy(bufi[b],
                            acc_sh.at[didxb.at[pl.ds(k * CSZ, CSZ)]],
                            add=True)
            if k + DEPTH < SB:
                hs[k + DEPTH] = _issue(k + DEPTH)
        return carry
    lax.fori_loop(0, NSB, _super, 0)

    plsc.subcore_barrier()
    pltpu.sync_copy(acc_sh.at[pl.ds(sid * ZR, ZR)],
                    outm_hbm.at[cid, pl.ds(sid * ZR, ZR)])


@functools.partial(
    pl.kernel,
    out_type=jax.ShapeDtypeStruct((NC, NPH, A2), jnp.float32),
    mesh=_mesh,
    compiler_params=_sc_params,
    scratch_types=(
        [pltpu.VMEM((CSZ, R2), jnp.float32)] * DEPTH
        + [pltpu.VMEM((CSZ, 8), jnp.float32)] * DEPTH
        + [
            pltpu.VMEM((SB * CSZ,), jnp.int32),
            pltpu.VMEM((SB * CSZ,), jnp.int32),
            pltpu.VMEM_SHARED((NPH, A2), jnp.float32),
        ]
        + [pltpu.SemaphoreType.DMA] * (2 * DEPTH)
    ),
)
def _l2_edges(tab_hbm, a2_hbm, esrc_hbm, edst_hbm, outm_hbm, *refs):
    bufi = refs[0:DEPTH]
    dbuf = refs[DEPTH:2 * DEPTH]
    sidxb, didxb, accm_sh = refs[2 * DEPTH:2 * DEPTH + 3]
    gsem = refs[2 * DEPTH + 3:2 * DEPTH + 3 + DEPTH]
    dsem = refs[2 * DEPTH + 3 + DEPTH:]
    cid = lax.axis_index("c")
    sid = lax.axis_index("s")
    ebase = sid * EPT
    rowbias = cid * N

    _zero_fill(bufi[0], CSZ, A2)
    _zero_acc_rows(accm_sh, bufi[0], sid * ZR)
    plsc.subcore_barrier()

    iota = lax.iota(jnp.int32, 16)
    zero16 = jnp.zeros((16,), jnp.int32)

    def _super(s, carry):
        eb = ebase + s * (SB * CSZ)
        pltpu.sync_copy(esrc_hbm.at[pl.ds(eb, SB * CSZ)], sidxb)
        pltpu.sync_copy(edst_hbm.at[pl.ds(eb, SB * CSZ)], didxb)

        def _bias(g, c2):
            sl = pl.ds(g * 16, 16)
            sidxb[sl] = sidxb[sl] + rowbias
            return c2
        lax.fori_loop(0, SB * CSZ // 16, _bias, 0)

        def _issue(k):
            sl = pl.ds(k * CSZ, CSZ)
            b = k % DEPTH
            return (
                pltpu.async_copy(tab_hbm.at[sidxb.at[sl]], bufi[b], gsem[b]),
                pltpu.async_copy(a2_hbm.at[didxb.at[sl]], dbuf[b], dsem[b]),
            )

        hs = {k: _issue(k) for k in range(DEPTH)}
        for k in range(SB):
            ha, hb = hs.pop(k)
            ha.wait()
            hb.wait()
            b = k % DEPTH
            for g in range(CSZ // 16):
                e16 = iota + g * 16
                a_s = plsc.load_gather(bufi[b], [e16, zero16])
                a_d = plsc.load_gather(dbuf[b], [e16, zero16])
                ew = a_s + a_d
                ew = jnp.where(ew > 0.0, ew, 0.2 * ew)
                w = jnp.exp(ew)
                plsc.store_scatter(bufi[b], [e16, zero16], w)

            def _edge(q, carry2, b=b):
                for u in range(4):
                    i = q * 4 + u
                    ws = bufi[b][i, pl.ds(0, 16)][0]
                    for kk in range((A2 - 16) // 16):
                        sl = pl.ds(16 + kk * 16, 16)
                        bufi[b][i, sl] = bufi[b][i, sl] * ws
                return carry2
            lax.fori_loop(0, CSZ // 4, _edge, 0)
            pltpu.sync_copy(bufi[b],
                            accm_sh.at[didxb.at[pl.ds(k * CSZ, CSZ)]],
                            add=True)
            if k + DEPTH < SB:
                hs[k + DEPTH] = _issue(k + DEPTH)
        return carry
    lax.fori_loop(0, NSB, _super, 0)

    plsc.subcore_barrier()
    pltpu.sync_copy(accm_sh.at[pl.ds(sid * ZR, ZR)],
                    outm_hbm.at[cid, pl.ds(sid * ZR, ZR)])


BR = 400
NB = N // BR


def _tc_pre_body(x_ref, w_ref, as_ref, ad_ref,
                 taba_ref, tabb_ref, adsta_ref, adstb_ref):
    h = jnp.dot(x_ref[...], w_ref[...], preferred_element_type=jnp.float32)
    hr = h.reshape(BR, H1, C1)
    asrc = (hr * as_ref[...][None]).sum(-1)
    adst = (hr * ad_ref[...][None]).sum(-1)
    z12 = jnp.zeros((BR, 12), jnp.float32)
    z4 = jnp.zeros((BR, 4), jnp.float32)
    taba_ref[...] = jnp.concatenate(
        [asrc[:, 0:HH], z12, h[:, 0:HH * C1]], axis=1)
    tabb_ref[...] = jnp.concatenate(
        [asrc[:, HH:H1], z12, h[:, HH * C1:D_IN]], axis=1)
    adsta_ref[...] = jnp.concatenate([adst[:, 0:HH], z4], axis=1)
    adstb_ref[...] = jnp.concatenate([adst[:, HH:H1], z4], axis=1)


def _tc_mid_body(accm_ref, b1_ref, w2_ref, as2_ref, ad2_ref,
                 tab2a_ref, tab2b_ref, a2tab_ref):
    wsum = jnp.concatenate(
        [accm_ref[0, :, 0:HH], accm_ref[1, :, 0:HH]], axis=1)
    msg = jnp.concatenate(
        [accm_ref[0, :, 16:A1], accm_ref[1, :, 16:A1]], axis=1)
    o = msg.reshape(BR, H1, C1) / (wsum[:, :, None] + 1e-16)
    o = (o + b1_ref[...].reshape(1, H1, C1)).reshape(BR, D_IN)
    o = jnp.where(o > 0.0, o, jnp.exp(o) - 1.0)
    h2 = jnp.dot(o, w2_ref[...], preferred_element_type=jnp.float32)
    asrc2 = (h2 * as2_ref[...]).sum(-1, keepdims=True)
    adst2 = (h2 * ad2_ref[...]).sum(-1, keepdims=True)
    z15 = jnp.zeros((BR, 15), jnp.float32)
    tab2a_ref[...] = jnp.concatenate([asrc2, z15, h2[:, 0:CH]], axis=1)
    tab2b_ref[...] = jnp.concatenate([asrc2, z15, h2[:, CH:C2]], axis=1)
    a2tab_ref[...] = jnp.concatenate(
        [adst2, jnp.zeros((BR, 7), jnp.float32)], axis=1)


def _tc_post_body(accm_ref, b2_ref, out_ref):
    msg = jnp.concatenate(
        [accm_ref[0, :, 16:A2], accm_ref[1, :, 16:A2]], axis=1)
    w = accm_ref[0, :, 0:1]
    out_ref[...] = msg / (w + 1e-16) + b2_ref[...]


_pre_call = pl.pallas_call(
    _tc_pre_body,
    grid=(NB,),
    in_specs=[
        pl.BlockSpec((BR, D_IN), lambda i: (i, 0)),
        pl.BlockSpec((D_IN, H1 * C1), lambda i: (0, 0)),
        pl.BlockSpec((H1, C1), lambda i: (0, 0)),
        pl.BlockSpec((H1, C1), lambda i: (0, 0)),
    ],
    out_specs=[
        pl.BlockSpec((BR, R1), lambda i: (i, 0)),
        pl.BlockSpec((BR, R1), lambda i: (i, 0)),
        pl.BlockSpec((BR, 8), lambda i: (i, 0)),
        pl.BlockSpec((BR, 8), lambda i: (i, 0)),
    ],
    out_shape=[
        jax.ShapeDtypeStruct((N, R1), jnp.float32),
        jax.ShapeDtypeStruct((N, R1), jnp.float32),
        jax.ShapeDtypeStruct((N, 8), jnp.float32),
        jax.ShapeDtypeStruct((N, 8), jnp.float32),
    ],
)

_mid_call = pl.pallas_call(
    _tc_mid_body,
    grid=(NB,),
    in_specs=[
        pl.BlockSpec((NC, BR, A1), lambda i: (0, i, 0)),
        pl.BlockSpec((1, D_IN), lambda i: (0, 0)),
        pl.BlockSpec((D_IN, H2 * C2), lambda i: (0, 0)),
        pl.BlockSpec((1, C2), lambda i: (0, 0)),
        pl.BlockSpec((1, C2), lambda i: (0, 0)),
    ],
    out_specs=[
        pl.BlockSpec((BR, R2), lambda i: (i, 0)),
        pl.BlockSpec((BR, R2), lambda i: (i, 0)),
        pl.BlockSpec((BR, 8), lambda i: (i, 0)),
    ],
    out_shape=[
        jax.ShapeDtypeStruct((N, R2), jnp.float32),
        jax.ShapeDtypeStruct((N, R2), jnp.float32),
        jax.ShapeDtypeStruct((N, 8), jnp.float32),
    ],
)

_post_call = pl.pallas_call(
    _tc_post_body,
    grid=(NB,),
    in_specs=[
        pl.BlockSpec((NC, BR, A2), lambda i: (0, i, 0)),
        pl.BlockSpec((1, C2), lambda i: (0, 0)),
    ],
    out_specs=pl.BlockSpec((BR, C2), lambda i: (i, 0)),
    out_shape=jax.ShapeDtypeStruct((N, C2), jnp.float32),
)


def kernel(x, edge_index, W1, a_src1, a_dst1, b1, W2, a_src2, a_dst2, b2):
    esrc, edst = edge_index[0], edge_index[1]
    taba, tabb, adsta, adstb = _pre_call(x, W1, a_src1, a_dst1)
    tab1 = jnp.concatenate([taba, tabb], axis=0)
    adst1 = jnp.concatenate([adsta, adstb], axis=0)
    accm1 = _l1_edges(tab1, adst1, esrc, edst)
    accm1 = accm1[:, 0:N]
    tab2a, tab2b, a2tab = _mid_call(accm1, b1.reshape(1, D_IN),
                                    W2, a_src2.reshape(1, C2),
                                    a_dst2.reshape(1, C2))
    tab2 = jnp.concatenate([tab2a, tab2b], axis=0)
    accm2 = _l2_edges(tab2, a2tab, esrc, edst)
    accm2 = accm2[:, 0:N]
    return _post_call(accm2, b2.reshape(1, C2))


# speedup vs baseline: 66.0073x; 1.0855x over previous
import functools

import jax
import jax.numpy as jnp
from jax import lax
from jax.experimental import pallas as pl
from jax.experimental.pallas import tpu as pltpu
from jax.experimental.pallas import tpu_sc as plsc

N = 10000
E = 320000
D_IN = 128
H1, C1 = 8, 16
H2, C2 = 1, 64
HH = H1 // 2
CH = C2 // 2
R1 = 80
A1 = 80
R2 = 48
A2 = 48

NC, NS = 2, 16
EPT = E // NS
CSZ = 80
NCH = EPT // CSZ
SB = 10
NSB = NCH // SB
DEPTH = 4
NPH = 10112
ZR = NPH // NS

_mesh = plsc.VectorSubcoreMesh(core_axis_name="c", subcore_axis_name="s")
_sc_params = pltpu.CompilerParams(use_tc_tiling_on_sc=False,
                                  needs_layout_passes=False)


def _zero_fill(buf, rows, width):
    def _zrow(r, carry):
        for k in range(width // 16):
            buf[r, pl.ds(k * 16, 16)] = jnp.zeros((16,), jnp.float32)
        return carry
    lax.fori_loop(0, rows, _zrow, 0)


def _zero_acc_rows(acc, zbuf, zbase):
    for t in range(ZR // CSZ):
        pltpu.sync_copy(zbuf, acc.at[pl.ds(zbase + t * CSZ, CSZ)])
    rem = ZR - (ZR // CSZ) * CSZ
    if rem:
        pltpu.sync_copy(zbuf.at[pl.ds(0, rem)],
                        acc.at[pl.ds(zbase + ZR - rem, rem)])


@functools.partial(
    pl.kernel,
    out_type=jax.ShapeDtypeStruct((NC, NPH, A1), jnp.float32),
    mesh=_mesh,
    compiler_params=_sc_params,
    scratch_types=(
        [pltpu.VMEM((CSZ, R1), jnp.float32)] * DEPTH
        + [pltpu.VMEM((CSZ, 8), jnp.float32)] * DEPTH
        + [
            pltpu.VMEM((SB * CSZ,), jnp.int32),
            pltpu.VMEM((SB * CSZ,), jnp.int32),
            pltpu.VMEM((SB * CSZ,), jnp.int32),
            pltpu.VMEM_SHARED((NPH, A1), jnp.float32),
        ]
        + [pltpu.SemaphoreType.DMA] * (2 * DEPTH)
    ),
)
def _l1_edges(tab_hbm, adst_hbm, esrc_hbm, edst_hbm, outm_hbm, *refs):
    bufi = refs[0:DEPTH]
    dbuf = refs[DEPTH:2 * DEPTH]
    sidxb, didxb, didx2b, acc_sh = refs[2 * DEPTH:2 * DEPTH + 4]
    gsem = refs[2 * DEPTH + 4:2 * DEPTH + 4 + DEPTH]
    dsem = refs[2 * DEPTH + 4 + DEPTH:]
    cid = lax.axis_index("c")
    sid = lax.axis_index("s")
    ebase = sid * EPT
    rowbias = cid

    _zero_fill(bufi[0], CSZ, A1)
    _zero_acc_rows(acc_sh, bufi[0], sid * ZR)
    plsc.subcore_barrier()

    iota = lax.iota(jnp.int32, 16)

    def _super(s, carry):
        eb = ebase + s * (SB * CSZ)
        pltpu.sync_copy(esrc_hbm.at[pl.ds(eb, SB * CSZ)], sidxb)
        pltpu.sync_copy(edst_hbm.at[pl.ds(eb, SB * CSZ)], didxb)

        def _bias(g, c2):
            sl = pl.ds(g * 16, 16)
            sidxb[sl] = sidxb[sl] * 2 + rowbias
            didx2b[sl] = didxb[sl] * 2 + rowbias
            return c2
        lax.fori_loop(0, SB * CSZ // 16, _bias, 0)

        def _issue(k):
            sl = pl.ds(k * CSZ, CSZ)
            b = k % DEPTH
            return (
                pltpu.async_copy(tab_hbm.at[sidxb.at[sl]], bufi[b], gsem[b]),
                pltpu.async_copy(adst_hbm.at[didx2b.at[sl]], dbuf[b], dsem[b]),
            )

        hs = {k: _issue(k) for k in range(DEPTH)}
        for k in range(SB):
            ha, hb = hs.pop(k)
            ha.wait()
            hb.wait()
            b = k % DEPTH
            for g in range(CSZ // 16):
                e16 = iota + g * 16
                for hd in range(HH):
                    hdv = jnp.full((16,), hd, jnp.int32)
                    a_s = plsc.load_gather(bufi[b], [e16, hdv])
                    a_d = plsc.load_gather(dbuf[b], [e16, hdv])
                    ew = a_s + a_d
                    ew = jnp.where(ew > 0.0, ew, 0.2 * ew)
                    w = jnp.exp(ew)
                    plsc.store_scatter(bufi[b], [e16, hdv], w)

            def _edge(q, carry2, b=b):
                for u in range(4):
                    i = q * 4 + u
                    wv = bufi[b][i, pl.ds(0, 16)]
                    for hd in range(HH):
                        ws = wv[hd]
                        sl = pl.ds(16 + hd * 16, 16)
                        bufi[b][i, sl] = bufi[b][i, sl] * ws
                return carry2
            lax.fori_loop(0, CSZ // 4, _edge, 0)
            pltpu.sync_copy(bufi[b],
                            acc_sh.at[didxb.at[pl.ds(k * CSZ, CSZ)]],
                            add=True)
            if k + DEPTH < SB:
                hs[k + DEPTH] = _issue(k + DEPTH)
        return carry
    lax.fori_loop(0, NSB, _super, 0)

    plsc.subcore_barrier()
    pltpu.sync_copy(acc_sh.at[pl.ds(sid * ZR, ZR)],
                    outm_hbm.at[cid, pl.ds(sid * ZR, ZR)])


@functools.partial(
    pl.kernel,
    out_type=jax.ShapeDtypeStruct((NC, NPH, A2), jnp.float32),
    mesh=_mesh,
    compiler_params=_sc_params,
    scratch_types=(
        [pltpu.VMEM((CSZ, R2), jnp.float32)] * DEPTH
        + [pltpu.VMEM((CSZ, 8), jnp.float32)] * DEPTH
        + [
            pltpu.VMEM((SB * CSZ,), jnp.int32),
            pltpu.VMEM((SB * CSZ,), jnp.int32),
            pltpu.VMEM_SHARED((NPH, A2), jnp.float32),
        ]
        + [pltpu.SemaphoreType.DMA] * (2 * DEPTH)
    ),
)
def _l2_edges(tab_hbm, a2_hbm, esrc_hbm, edst_hbm, outm_hbm, *refs):
    bufi = refs[0:DEPTH]
    dbuf = refs[DEPTH:2 * DEPTH]
    sidxb, didxb, accm_sh = refs[2 * DEPTH:2 * DEPTH + 3]
    gsem = refs[2 * DEPTH + 3:2 * DEPTH + 3 + DEPTH]
    dsem = refs[2 * DEPTH + 3 + DEPTH:]
    cid = lax.axis_index("c")
    sid = lax.axis_index("s")
    ebase = sid * EPT
    rowbias = cid

    _zero_fill(bufi[0], CSZ, A2)
    _zero_acc_rows(accm_sh, bufi[0], sid * ZR)
    plsc.subcore_barrier()

    iota = lax.iota(jnp.int32, 16)
    zero16 = jnp.zeros((16,), jnp.int32)

    def _super(s, carry):
        eb = ebase + s * (SB * CSZ)
        pltpu.sync_copy(esrc_hbm.at[pl.ds(eb, SB * CSZ)], sidxb)
        pltpu.sync_copy(edst_hbm.at[pl.ds(eb, SB * CSZ)], didxb)

        def _bias(g, c2):
            sl = pl.ds(g * 16, 16)
            sidxb[sl] = sidxb[sl] * 2 + rowbias
            return c2
        lax.fori_loop(0, SB * CSZ // 16, _bias, 0)

        def _issue(k):
            sl = pl.ds(k * CSZ, CSZ)
            b = k % DEPTH
            return (
                pltpu.async_copy(tab_hbm.at[sidxb.at[sl]], bufi[b], gsem[b]),
                pltpu.async_copy(a2_hbm.at[didxb.at[sl]], dbuf[b], dsem[b]),
            )

        hs = {k: _issue(k) for k in range(DEPTH)}
        for k in range(SB):
            ha, hb = hs.pop(k)
            ha.wait()
            hb.wait()
            b = k % DEPTH
            for g in range(CSZ // 16):
                e16 = iota + g * 16
                a_s = plsc.load_gather(bufi[b], [e16, zero16])
                a_d = plsc.load_gather(dbuf[b], [e16, zero16])
                ew = a_s + a_d
                ew = jnp.where(ew > 0.0, ew, 0.2 * ew)
                w = jnp.exp(ew)
                plsc.store_scatter(bufi[b], [e16, zero16], w)

            def _edge(q, carry2, b=b):
                for u in range(4):
                    i = q * 4 + u
                    ws = bufi[b][i, pl.ds(0, 16)][0]
                    for kk in range((A2 - 16) // 16):
                        sl = pl.ds(16 + kk * 16, 16)
                        bufi[b][i, sl] = bufi[b][i, sl] * ws
                return carry2
            lax.fori_loop(0, CSZ // 4, _edge, 0)
            pltpu.sync_copy(bufi[b],
                            accm_sh.at[didxb.at[pl.ds(k * CSZ, CSZ)]],
                            add=True)
            if k + DEPTH < SB:
                hs[k + DEPTH] = _issue(k + DEPTH)
        return carry
    lax.fori_loop(0, NSB, _super, 0)

    plsc.subcore_barrier()
    pltpu.sync_copy(accm_sh.at[pl.ds(sid * ZR, ZR)],
                    outm_hbm.at[cid, pl.ds(sid * ZR, ZR)])


BR = 400
NB = N // BR


def _tc_pre_body(x_ref, w_ref, as_ref, ad_ref, tab_ref, adst_ref):
    h = jnp.dot(x_ref[...], w_ref[...], preferred_element_type=jnp.float32)
    hr = h.reshape(BR, H1, C1)
    asrc = (hr * as_ref[...][None]).sum(-1)
    adst = (hr * ad_ref[...][None]).sum(-1)
    z12 = jnp.zeros((BR, 12), jnp.float32)
    z4 = jnp.zeros((BR, 4), jnp.float32)
    tab_ref[...] = jnp.concatenate(
        [asrc[:, 0:HH], z12, h[:, 0:HH * C1],
         asrc[:, HH:H1], z12, h[:, HH * C1:D_IN]], axis=1)
    adst_ref[...] = jnp.concatenate(
        [adst[:, 0:HH], z4, adst[:, HH:H1], z4], axis=1)


def _tc_mid_body(accm_ref, b1_ref, w2_ref, as2_ref, ad2_ref,
                 tab2_ref, a2tab_ref):
    wsum = jnp.concatenate(
        [accm_ref[0, :, 0:HH], accm_ref[1, :, 0:HH]], axis=1)
    msg = jnp.concatenate(
        [accm_ref[0, :, 16:A1], accm_ref[1, :, 16:A1]], axis=1)
    o = msg.reshape(BR, H1, C1) / (wsum[:, :, None] + 1e-16)
    o = (o + b1_ref[...].reshape(1, H1, C1)).reshape(BR, D_IN)
    o = jnp.where(o > 0.0, o, jnp.exp(o) - 1.0)
    h2 = jnp.dot(o, w2_ref[...], preferred_element_type=jnp.float32)
    asrc2 = (h2 * as2_ref[...]).sum(-1, keepdims=True)
    adst2 = (h2 * ad2_ref[...]).sum(-1, keepdims=True)
    z15 = jnp.zeros((BR, 15), jnp.float32)
    tab2_ref[...] = jnp.concatenate(
        [asrc2, z15, h2[:, 0:CH], asrc2, z15, h2[:, CH:C2]], axis=1)
    a2tab_ref[...] = jnp.concatenate(
        [adst2, jnp.zeros((BR, 7), jnp.float32)], axis=1)


def _tc_post_body(accm_ref, b2_ref, out_ref):
    msg = jnp.concatenate(
        [accm_ref[0, :, 16:A2], accm_ref[1, :, 16:A2]], axis=1)
    w = accm_ref[0, :, 0:1]
    out_ref[...] = msg / (w + 1e-16) + b2_ref[...]


_pre_call = pl.pallas_call(
    _tc_pre_body,
    grid=(NB,),
    in_specs=[
        pl.BlockSpec((BR, D_IN), lambda i: (i, 0)),
        pl.BlockSpec((D_IN, H1 * C1), lambda i: (0, 0)),
        pl.BlockSpec((H1, C1), lambda i: (0, 0)),
        pl.BlockSpec((H1, C1), lambda i: (0, 0)),
    ],
    out_specs=[
        pl.BlockSpec((BR, 2 * R1), lambda i: (i, 0)),
        pl.BlockSpec((BR, 16), lambda i: (i, 0)),
    ],
    out_shape=[
        jax.ShapeDtypeStruct((N, 2 * R1), jnp.float32),
        jax.ShapeDtypeStruct((N, 16), jnp.float32),
    ],
)

_mid_call = pl.pallas_call(
    _tc_mid_body,
    grid=(NB,),
    in_specs=[
        pl.BlockSpec((NC, BR, A1), lambda i: (0, i, 0)),
        pl.BlockSpec((1, D_IN), lambda i: (0, 0)),
        pl.BlockSpec((D_IN, H2 * C2), lambda i: (0, 0)),
        pl.BlockSpec((1, C2), lambda i: (0, 0)),
        pl.BlockSpec((1, C2), lambda i: (0, 0)),
    ],
    out_specs=[
        pl.BlockSpec((BR, 2 * R2), lambda i: (i, 0)),
        pl.BlockSpec((BR, 8), lambda i: (i, 0)),
    ],
    out_shape=[
        jax.ShapeDtypeStruct((N, 2 * R2), jnp.float32),
        jax.ShapeDtypeStruct((N, 8), jnp.float32),
    ],
)

_post_call = pl.pallas_call(
    _tc_post_body,
    grid=(NB,),
    in_specs=[
        pl.BlockSpec((NC, BR, A2), lambda i: (0, i, 0)),
        pl.BlockSpec((1, C2), lambda i: (0, 0)),
    ],
    out_specs=pl.BlockSpec((BR, C2), lambda i: (i, 0)),
    out_shape=jax.ShapeDtypeStruct((N, C2), jnp.float32),
)


def kernel(x, edge_index, W1, a_src1, a_dst1, b1, W2, a_src2, a_dst2, b2):
    esrc, edst = edge_index[0], edge_index[1]
    tab1w, adst1w = _pre_call(x, W1, a_src1, a_dst1)
    tab1 = tab1w.reshape(2 * N, R1)
    adst1 = adst1w.reshape(2 * N, 8)
    accm1 = _l1_edges(tab1, adst1, esrc, edst)
    tab2w, a2tab = _mid_call(accm1, b1.reshape(1, D_IN),
                             W2, a_src2.reshape(1, C2),
                             a_dst2.reshape(1, C2))
    tab2 = tab2w.reshape(2 * N, R2)
    accm2 = _l2_edges(tab2, a2tab, esrc, edst)
    return _post_call(accm2, b2.reshape(1, C2))


# speedup vs baseline: 72.2395x; 1.0944x over previous
import functools

import jax
import jax.numpy as jnp
from jax import lax
from jax.experimental import pallas as pl
from jax.experimental.pallas import tpu as pltpu
from jax.experimental.pallas import tpu_sc as plsc

N = 10000
E = 320000
D_IN = 128
H1, C1 = 8, 16
H2, C2 = 1, 64
HH = H1 // 2
CH = C2 // 2
R1 = 80
A1 = 80
R2 = 80
A2 = 80

NC, NS = 2, 16
EPT = E // NS
CSZ = 80
NCH = EPT // CSZ
SB = 10
NSB = NCH // SB
DEPTH = 4
EPT2 = E // (NC * NS)
NCH2 = EPT2 // CSZ
SB2 = 5
NSB2 = NCH2 // SB2
NPH = 10112
ZR = NPH // NS

_mesh = plsc.VectorSubcoreMesh(core_axis_name="c", subcore_axis_name="s")
_sc_params = pltpu.CompilerParams(use_tc_tiling_on_sc=False,
                                  needs_layout_passes=False)


def _zero_fill(buf, rows, width):
    def _zrow(r, carry):
        for k in range(width // 16):
            buf[r, pl.ds(k * 16, 16)] = jnp.zeros((16,), jnp.float32)
        return carry
    lax.fori_loop(0, rows, _zrow, 0)


def _zero_acc_rows(acc, zbuf, zbase):
    for t in range(ZR // CSZ):
        pltpu.sync_copy(zbuf, acc.at[pl.ds(zbase + t * CSZ, CSZ)])
    rem = ZR - (ZR // CSZ) * CSZ
    if rem:
        pltpu.sync_copy(zbuf.at[pl.ds(0, rem)],
                        acc.at[pl.ds(zbase + ZR - rem, rem)])


@functools.partial(
    pl.kernel,
    out_type=jax.ShapeDtypeStruct((NC, NPH, A1), jnp.float32),
    mesh=_mesh,
    compiler_params=_sc_params,
    scratch_types=(
        [pltpu.VMEM((CSZ, R1), jnp.float32)] * DEPTH
        + [pltpu.VMEM((CSZ, 8), jnp.float32)] * DEPTH
        + [
            pltpu.VMEM((SB * CSZ,), jnp.int32),
            pltpu.VMEM((SB * CSZ,), jnp.int32),
            pltpu.VMEM((SB * CSZ,), jnp.int32),
            pltpu.VMEM_SHARED((NPH, A1), jnp.float32),
        ]
        + [pltpu.SemaphoreType.DMA] * (2 * DEPTH)
    ),
)
def _l1_edges(tab_hbm, adst_hbm, esrc_hbm, edst_hbm, outm_hbm, *refs):
    bufi = refs[0:DEPTH]
    dbuf = refs[DEPTH:2 * DEPTH]
    sidxb, didxb, didx2b, acc_sh = refs[2 * DEPTH:2 * DEPTH + 4]
    gsem = refs[2 * DEPTH + 4:2 * DEPTH + 4 + DEPTH]
    dsem = refs[2 * DEPTH + 4 + DEPTH:]
    cid = lax.axis_index("c")
    sid = lax.axis_index("s")
    ebase = sid * EPT
    rowbias = cid

    _zero_fill(bufi[0], CSZ, A1)
    _zero_acc_rows(acc_sh, bufi[0], sid * ZR)
    plsc.subcore_barrier()

    iota = lax.iota(jnp.int32, 16)

    def _super(s, carry):
        eb = ebase + s * (SB * CSZ)
        pltpu.sync_copy(esrc_hbm.at[pl.ds(eb, SB * CSZ)], sidxb)
        pltpu.sync_copy(edst_hbm.at[pl.ds(eb, SB * CSZ)], didxb)

        def _bias(g, c2):
            sl = pl.ds(g * 16, 16)
            sidxb[sl] = sidxb[sl] * 2 + rowbias
            didx2b[sl] = didxb[sl] * 2 + rowbias
            return c2
        lax.fori_loop(0, SB * CSZ // 16, _bias, 0)

        def _issue(k):
            sl = pl.ds(k * CSZ, CSZ)
            b = k % DEPTH
            return (
                pltpu.async_copy(tab_hbm.at[sidxb.at[sl]], bufi[b], gsem[b]),
                pltpu.async_copy(adst_hbm.at[didx2b.at[sl]], dbuf[b], dsem[b]),
            )

        hs = {k: _issue(k) for k in range(DEPTH)}
        for k in range(SB):
            ha, hb = hs.pop(k)
            ha.wait()
            hb.wait()
            b = k % DEPTH
            for g in range(CSZ // 16):
                e16 = iota + g * 16
                for hd in range(HH):
                    hdv = jnp.full((16,), hd, jnp.int32)
                    a_s = plsc.load_gather(bufi[b], [e16, hdv])
                    a_d = plsc.load_gather(dbuf[b], [e16, hdv])
                    ew = a_s + a_d
                    ew = jnp.where(ew > 0.0, ew, 0.2 * ew)
                    w = jnp.exp(ew)
                    plsc.store_scatter(bufi[b], [e16, hdv], w)

            def _edge(q, carry2, b=b):
                for u in range(4):
                    i = q * 4 + u
                    wv = bufi[b][i, pl.ds(0, 16)]
                    for hd in range(HH):
                        ws = wv[hd]
                        sl = pl.ds(16 + hd * 16, 16)
                        bufi[b][i, sl] = bufi[b][i, sl] * ws
                return carry2
            lax.fori_loop(0, CSZ // 4, _edge, 0)
            pltpu.sync_copy(bufi[b],
                            acc_sh.at[didxb.at[pl.ds(k * CSZ, CSZ)]],
                            add=True)
            if k + DEPTH < SB:
                hs[k + DEPTH] = _issue(k + DEPTH)
        return carry
    lax.fori_loop(0, NSB, _super, 0)

    plsc.subcore_barrier()
    pltpu.sync_copy(acc_sh.at[pl.ds(sid * ZR, ZR)],
                    outm_hbm.at[cid, pl.ds(sid * ZR, ZR)])


@functools.partial(
    pl.kernel,
    out_type=jax.ShapeDtypeStruct((NC, NPH, A2), jnp.float32),
    mesh=_mesh,
    compiler_params=_sc_params,
    scratch_types=(
        [pltpu.VMEM((CSZ, R2), jnp.float32)] * DEPTH
        + [pltpu.VMEM((CSZ, 8), jnp.float32)] * DEPTH
        + [
            pltpu.VMEM((SB2 * CSZ,), jnp.int32),
            pltpu.VMEM((SB2 * CSZ,), jnp.int32),
            pltpu.VMEM_SHARED((NPH, A2), jnp.float32),
        ]
        + [pltpu.SemaphoreType.DMA] * (2 * DEPTH)
    ),
)
def _l2_edges(tab_hbm, a2_hbm, esrc_hbm, edst_hbm, outm_hbm, *refs):
    bufi = refs[0:DEPTH]
    dbuf = refs[DEPTH:2 * DEPTH]
    sidxb, didxb, accm_sh = refs[2 * DEPTH:2 * DEPTH + 3]
    gsem = refs[2 * DEPTH + 3:2 * DEPTH + 3 + DEPTH]
    dsem = refs[2 * DEPTH + 3 + DEPTH:]
    cid = lax.axis_index("c")
    sid = lax.axis_index("s")
    ebase = (cid * NS + sid) * EPT2

    _zero_fill(bufi[0], CSZ, A2)
    _zero_acc_rows(accm_sh, bufi[0], sid * ZR)
    plsc.subcore_barrier()

    iota = lax.iota(jnp.int32, 16)
    zero16 = jnp.zeros((16,), jnp.int32)

    def _super(s, carry):
        eb = ebase + s * (SB2 * CSZ)
        pltpu.sync_copy(esrc_hbm.at[pl.ds(eb, SB2 * CSZ)], sidxb)
        pltpu.sync_copy(edst_hbm.at[pl.ds(eb, SB2 * CSZ)], didxb)

        def _issue(k):
            sl = pl.ds(k * CSZ, CSZ)
            b = k % DEPTH
            return (
                pltpu.async_copy(tab_hbm.at[sidxb.at[sl]], bufi[b], gsem[b]),
                pltpu.async_copy(a2_hbm.at[didxb.at[sl]], dbuf[b], dsem[b]),
            )

        hs = {k: _issue(k) for k in range(min(DEPTH, SB2))}
        for k in range(SB2):
            ha, hb = hs.pop(k)
            ha.wait()
            hb.wait()
            b = k % DEPTH
            for g in range(CSZ // 16):
                e16 = iota + g * 16
                a_s = plsc.load_gather(bufi[b], [e16, zero16])
                a_d = plsc.load_gather(dbuf[b], [e16, zero16])
                ew = a_s + a_d
                ew = jnp.where(ew > 0.0, ew, 0.2 * ew)
                w = jnp.exp(ew)
                plsc.store_scatter(bufi[b], [e16, zero16], w)

            def _edge(q, carry2, b=b):
                for u in range(4):
                    i = q * 4 + u
                    ws = bufi[b][i, pl.ds(0, 16)][0]
                    for kk in range((A2 - 16) // 16):
                        sl = pl.ds(16 + kk * 16, 16)
                        bufi[b][i, sl] = bufi[b][i, sl] * ws
                return carry2
            lax.fori_loop(0, CSZ // 4, _edge, 0)
            pltpu.sync_copy(bufi[b],
                            accm_sh.at[didxb.at[pl.ds(k * CSZ, CSZ)]],
                            add=True)
            if k + DEPTH < SB2:
                hs[k + DEPTH] = _issue(k + DEPTH)
        return carry
    lax.fori_loop(0, NSB2, _super, 0)

    plsc.subcore_barrier()
    pltpu.sync_copy(accm_sh.at[pl.ds(sid * ZR, ZR)],
                    outm_hbm.at[cid, pl.ds(sid * ZR, ZR)])


BR = 400
NB = N // BR


def _tc_pre_body(x_ref, w_ref, as_ref, ad_ref, tab_ref, adst_ref):
    h = jnp.dot(x_ref[...], w_ref[...], preferred_element_type=jnp.float32)
    hr = h.reshape(BR, H1, C1)
    asrc = (hr * as_ref[...][None]).sum(-1)
    adst = (hr * ad_ref[...][None]).sum(-1)
    z12 = jnp.zeros((BR, 12), jnp.float32)
    z4 = jnp.zeros((BR, 4), jnp.float32)
    tab_ref[...] = jnp.concatenate(
        [asrc[:, 0:HH], z12, h[:, 0:HH * C1],
         asrc[:, HH:H1], z12, h[:, HH * C1:D_IN]], axis=1)
    adst_ref[...] = jnp.concatenate(
        [adst[:, 0:HH], z4, adst[:, HH:H1], z4], axis=1)


def _tc_mid_body(accm_ref, b1_ref, w2_ref, as2_ref, ad2_ref,
                 tab2_ref, a2tab_ref):
    wsum = jnp.concatenate(
        [accm_ref[0, :, 0:HH], accm_ref[1, :, 0:HH]], axis=1)
    msg = jnp.concatenate(
        [accm_ref[0, :, 16:A1], accm_ref[1, :, 16:A1]], axis=1)
    o = msg.reshape(BR, H1, C1) / (wsum[:, :, None] + 1e-16)
    o = (o + b1_ref[...].reshape(1, H1, C1)).reshape(BR, D_IN)
    o = jnp.where(o > 0.0, o, jnp.exp(o) - 1.0)
    h2 = jnp.dot(o, w2_ref[...], preferred_element_type=jnp.float32)
    asrc2 = (h2 * as2_ref[...]).sum(-1, keepdims=True)
    adst2 = (h2 * ad2_ref[...]).sum(-1, keepdims=True)
    z15 = jnp.zeros((BR, 15), jnp.float32)
    tab2_ref[...] = jnp.concatenate([asrc2, z15, h2], axis=1)
    a2tab_ref[...] = jnp.concatenate(
        [adst2, jnp.zeros((BR, 7), jnp.float32)], axis=1)


def _tc_post_body(accm_ref, b2_ref, out_ref):
    msg = accm_ref[0, :, 16:A2] + accm_ref[1, :, 16:A2]
    w = accm_ref[0, :, 0:1] + accm_ref[1, :, 0:1]
    out_ref[...] = msg / (w + 1e-16) + b2_ref[...]


_pre_call = pl.pallas_call(
    _tc_pre_body,
    grid=(NB,),
    in_specs=[
        pl.BlockSpec((BR, D_IN), lambda i: (i, 0)),
        pl.BlockSpec((D_IN, H1 * C1), lambda i: (0, 0)),
        pl.BlockSpec((H1, C1), lambda i: (0, 0)),
        pl.BlockSpec((H1, C1), lambda i: (0, 0)),
    ],
    out_specs=[
        pl.BlockSpec((BR, 2 * R1), lambda i: (i, 0)),
        pl.BlockSpec((BR, 16), lambda i: (i, 0)),
    ],
    out_shape=[
        jax.ShapeDtypeStruct((N, 2 * R1), jnp.float32),
        jax.ShapeDtypeStruct((N, 16), jnp.float32),
    ],
)

_mid_call = pl.pallas_call(
    _tc_mid_body,
    grid=(NB,),
    in_specs=[
        pl.BlockSpec((NC, BR, A1), lambda i: (0, i, 0)),
        pl.BlockSpec((1, D_IN), lambda i: (0, 0)),
        pl.BlockSpec((D_IN, H2 * C2), lambda i: (0, 0)),
        pl.BlockSpec((1, C2), lambda i: (0, 0)),
        pl.BlockSpec((1, C2), lambda i: (0, 0)),
    ],
    out_specs=[
        pl.BlockSpec((BR, R2), lambda i: (i, 0)),
        pl.BlockSpec((BR, 8), lambda i: (i, 0)),
    ],
    out_shape=[
        jax.ShapeDtypeStruct((N, R2), jnp.float32),
        jax.ShapeDtypeStruct((N, 8), jnp.float32),
    ],
)

_post_call = pl.pallas_call(
    _tc_post_body,
    grid=(NB,),
    in_specs=[
        pl.BlockSpec((NC, BR, A2), lambda i: (0, i, 0)),
        pl.BlockSpec((1, C2), lambda i: (0, 0)),
    ],
    out_specs=pl.BlockSpec((BR, C2), lambda i: (i, 0)),
    out_shape=jax.ShapeDtypeStruct((N, C2), jnp.float32),
)


def kernel(x, edge_index, W1, a_src1, a_dst1, b1, W2, a_src2, a_dst2, b2):
    esrc, edst = edge_index[0], edge_index[1]
    tab1w, adst1w = _pre_call(x, W1, a_src1, a_dst1)
    tab1 = tab1w.reshape(2 * N, R1)
    adst1 = adst1w.reshape(2 * N, 8)
    accm1 = _l1_edges(tab1, adst1, esrc, edst)
    tab2, a2tab = _mid_call(accm1, b1.reshape(1, D_IN),
                            W2, a_src2.reshape(1, C2),
                            a_dst2.reshape(1, C2))
    accm2 = _l2_edges(tab2, a2tab, esrc, edst)
    return _post_call(accm2, b2.reshape(1, C2))


# speedup vs baseline: 73.6095x; 1.0190x over previous
import functools

import jax
import jax.numpy as jnp
from jax import lax
from jax.experimental import pallas as pl
from jax.experimental.pallas import tpu as pltpu
from jax.experimental.pallas import tpu_sc as plsc

N = 10000
E = 320000
D_IN = 128
H1, C1 = 8, 16
H2, C2 = 1, 64
HH = H1 // 2
CH = C2 // 2
R1 = 144
A1 = 144
D1 = 2
R2 = 80
A2 = 80

NC, NS = 2, 16
EPT = E // NS
CSZ = 80
NCH = EPT // CSZ
SB = 10
NSB = NCH // SB
DEPTH = 4
EPT2 = E // (NC * NS)
NCH2 = EPT2 // CSZ
SB2 = 5
NSB2 = NCH2 // SB2
NPH = 10112
ZR = NPH // NS

_mesh = plsc.VectorSubcoreMesh(core_axis_name="c", subcore_axis_name="s")
_sc_params = pltpu.CompilerParams(use_tc_tiling_on_sc=False,
                                  needs_layout_passes=False)


def _zero_fill(buf, rows, width):
    def _zrow(r, carry):
        for k in range(width // 16):
            buf[r, pl.ds(k * 16, 16)] = jnp.zeros((16,), jnp.float32)
        return carry
    lax.fori_loop(0, rows, _zrow, 0)


def _zero_acc_rows(acc, zbuf, zbase):
    for t in range(ZR // CSZ):
        pltpu.sync_copy(zbuf, acc.at[pl.ds(zbase + t * CSZ, CSZ)])
    rem = ZR - (ZR // CSZ) * CSZ
    if rem:
        pltpu.sync_copy(zbuf.at[pl.ds(0, rem)],
                        acc.at[pl.ds(zbase + ZR - rem, rem)])


@functools.partial(
    pl.kernel,
    out_type=jax.ShapeDtypeStruct((NC, NPH, A1), jnp.float32),
    mesh=_mesh,
    compiler_params=_sc_params,
    scratch_types=(
        [pltpu.VMEM((CSZ, R1), jnp.float32)] * D1
        + [pltpu.VMEM((CSZ, 8), jnp.float32)] * D1
        + [
            pltpu.VMEM((SB2 * CSZ,), jnp.int32),
            pltpu.VMEM((SB2 * CSZ,), jnp.int32),
            pltpu.VMEM_SHARED((NPH, A1), jnp.float32),
        ]
        + [pltpu.SemaphoreType.DMA] * (2 * D1)
    ),
)
def _l1_edges(tab_hbm, adst_hbm, esrc_hbm, edst_hbm, outm_hbm, *refs):
    bufi = refs[0:D1]
    dbuf = refs[D1:2 * D1]
    sidxb, didxb, acc_sh = refs[2 * D1:2 * D1 + 3]
    gsem = refs[2 * D1 + 3:2 * D1 + 3 + D1]
    dsem = refs[2 * D1 + 3 + D1:]
    cid = lax.axis_index("c")
    sid = lax.axis_index("s")
    ebase = (cid * NS + sid) * EPT2

    _zero_fill(bufi[0], CSZ, A1)
    _zero_acc_rows(acc_sh, bufi[0], sid * ZR)
    plsc.subcore_barrier()

    iota = lax.iota(jnp.int32, 16)

    def _super(s, carry):
        eb = ebase + s * (SB2 * CSZ)
        pltpu.sync_copy(esrc_hbm.at[pl.ds(eb, SB2 * CSZ)], sidxb)
        pltpu.sync_copy(edst_hbm.at[pl.ds(eb, SB2 * CSZ)], didxb)

        def _issue(k):
            sl = pl.ds(k * CSZ, CSZ)
            b = k % D1
            return (
                pltpu.async_copy(tab_hbm.at[sidxb.at[sl]], bufi[b], gsem[b]),
                pltpu.async_copy(adst_hbm.at[didxb.at[sl]], dbuf[b], dsem[b]),
            )

        hs = {k: _issue(k) for k in range(min(D1, SB2))}
        for k in range(SB2):
            ha, hb = hs.pop(k)
            ha.wait()
            hb.wait()
            b = k % D1
            for g in range(CSZ // 16):
                e16 = iota + g * 16
                for hd in range(H1):
                    hdv = jnp.full((16,), hd, jnp.int32)
                    a_s = plsc.load_gather(bufi[b], [e16, hdv])
                    a_d = plsc.load_gather(dbuf[b], [e16, hdv])
                    ew = a_s + a_d
                    ew = jnp.where(ew > 0.0, ew, 0.2 * ew)
                    w = jnp.exp(ew)
                    plsc.store_scatter(bufi[b], [e16, hdv], w)

            def _edge(q, carry2, b=b):
                for u in range(4):
                    i = q * 4 + u
                    wv = bufi[b][i, pl.ds(0, 16)]
                    for hd in range(H1):
                        ws = wv[hd]
                        sl = pl.ds(16 + hd * 16, 16)
                        bufi[b][i, sl] = bufi[b][i, sl] * ws
                return carry2
            lax.fori_loop(0, CSZ // 4, _edge, 0)
            pltpu.sync_copy(bufi[b],
                            acc_sh.at[didxb.at[pl.ds(k * CSZ, CSZ)]],
                            add=True)
            if k + D1 < SB2:
                hs[k + D1] = _issue(k + D1)
        return carry
    lax.fori_loop(0, NSB2, _super, 0)

    plsc.subcore_barrier()
    pltpu.sync_copy(acc_sh.at[pl.ds(sid * ZR, ZR)],
                    outm_hbm.at[cid, pl.ds(sid * ZR, ZR)])


@functools.partial(
    pl.kernel,
    out_type=jax.ShapeDtypeStruct((NC, NPH, A2), jnp.float32),
    mesh=_mesh,
    compiler_params=_sc_params,
    scratch_types=(
        [pltpu.VMEM((CSZ, R2), jnp.float32)] * DEPTH
        + [pltpu.VMEM((CSZ, 8), jnp.float32)] * DEPTH
        + [
            pltpu.VMEM((SB2 * CSZ,), jnp.int32),
            pltpu.VMEM((SB2 * CSZ,), jnp.int32),
            pltpu.VMEM_SHARED((NPH, A2), jnp.float32),
        ]
        + [pltpu.SemaphoreType.DMA] * (2 * DEPTH)
    ),
)
def _l2_edges(tab_hbm, a2_hbm, esrc_hbm, edst_hbm, outm_hbm, *refs):
    bufi = refs[0:DEPTH]
    dbuf = refs[DEPTH:2 * DEPTH]
    sidxb, didxb, accm_sh = refs[2 * DEPTH:2 * DEPTH + 3]
    gsem = refs[2 * DEPTH + 3:2 * DEPTH + 3 + DEPTH]
    dsem = refs[2 * DEPTH + 3 + DEPTH:]
    cid = lax.axis_index("c")
    sid = lax.axis_index("s")
    ebase = (cid * NS + sid) * EPT2

    _zero_fill(bufi[0], CSZ, A2)
    _zero_acc_rows(accm_sh, bufi[0], sid * ZR)
    plsc.subcore_barrier()

    iota = lax.iota(jnp.int32, 16)
    zero16 = jnp.zeros((16,), jnp.int32)

    def _super(s, carry):
        eb = ebase + s * (SB2 * CSZ)
        pltpu.sync_copy(esrc_hbm.at[pl.ds(eb, SB2 * CSZ)], sidxb)
        pltpu.sync_copy(edst_hbm.at[pl.ds(eb, SB2 * CSZ)], didxb)

        def _issue(k):
            sl = pl.ds(k * CSZ, CSZ)
            b = k % DEPTH
            return (
                pltpu.async_copy(tab_hbm.at[sidxb.at[sl]], bufi[b], gsem[b]),
                pltpu.async_copy(a2_hbm.at[didxb.at[sl]], dbuf[b], dsem[b]),
            )

        hs = {k: _issue(k) for k in range(min(DEPTH, SB2))}
        for k in range(SB2):
            ha, hb = hs.pop(k)
            ha.wait()
            hb.wait()
            b = k % DEPTH
            for g in range(CSZ // 16):
                e16 = iota + g * 16
                a_s = plsc.load_gather(bufi[b], [e16, zero16])
                a_d = plsc.load_gather(dbuf[b], [e16, zero16])
                ew = a_s + a_d
                ew = jnp.where(ew > 0.0, ew, 0.2 * ew)
                w = jnp.exp(ew)
                plsc.store_scatter(bufi[b], [e16, zero16], w)

            def _edge(q, carry2, b=b):
                for u in range(4):
                    i = q * 4 + u
                    ws = bufi[b][i, pl.ds(0, 16)][0]
                    for kk in range((A2 - 16) // 16):
                        sl = pl.ds(16 + kk * 16, 16)
                        bufi[b][i, sl] = bufi[b][i, sl] * ws
                return carry2
            lax.fori_loop(0, CSZ // 4, _edge, 0)
            pltpu.sync_copy(bufi[b],
                            accm_sh.at[didxb.at[pl.ds(k * CSZ, CSZ)]],
                            add=True)
            if k + DEPTH < SB2:
                hs[k + DEPTH] = _issue(k + DEPTH)
        return carry
    lax.fori_loop(0, NSB2, _super, 0)

    plsc.subcore_barrier()
    pltpu.sync_copy(accm_sh.at[pl.ds(sid * ZR, ZR)],
                    outm_hbm.at[cid, pl.ds(sid * ZR, ZR)])


BR = 400
NB = N // BR


def _tc_pre_body(x_ref, w_ref, as_ref, ad_ref, tab_ref, adst_ref):
    h = jnp.dot(x_ref[...], w_ref[...], preferred_element_type=jnp.float32)
    hr = h.reshape(BR, H1, C1)
    asrc = (hr * as_ref[...][None]).sum(-1)
    adst = (hr * ad_ref[...][None]).sum(-1)
    z8 = jnp.zeros((BR, 8), jnp.float32)
    tab_ref[...] = jnp.concatenate([asrc, z8, h], axis=1)
    adst_ref[...] = adst


def _tc_mid_body(accm_ref, b1_ref, w2_ref, as2_ref, ad2_ref,
                 tab2_ref, a2tab_ref):
    wsum = accm_ref[0, :, 0:H1] + accm_ref[1, :, 0:H1]
    msg = accm_ref[0, :, 16:A1] + accm_ref[1, :, 16:A1]
    o = msg.reshape(BR, H1, C1) / (wsum[:, :, None] + 1e-16)
    o = (o + b1_ref[...].reshape(1, H1, C1)).reshape(BR, D_IN)
    o = jnp.where(o > 0.0, o, jnp.exp(o) - 1.0)
    h2 = jnp.dot(o, w2_ref[...], preferred_element_type=jnp.float32)
    asrc2 = (h2 * as2_ref[...]).sum(-1, keepdims=True)
    adst2 = (h2 * ad2_ref[...]).sum(-1, keepdims=True)
    z15 = jnp.zeros((BR, 15), jnp.float32)
    tab2_ref[...] = jnp.concatenate([asrc2, z15, h2], axis=1)
    a2tab_ref[...] = jnp.concatenate(
        [adst2, jnp.zeros((BR, 7), jnp.float32)], axis=1)


def _tc_post_body(accm_ref, b2_ref, out_ref):
    msg = accm_ref[0, :, 16:A2] + accm_ref[1, :, 16:A2]
    w = accm_ref[0, :, 0:1] + accm_ref[1, :, 0:1]
    out_ref[...] = msg / (w + 1e-16) + b2_ref[...]


_pre_call = pl.pallas_call(
    _tc_pre_body,
    grid=(NB,),
    in_specs=[
        pl.BlockSpec((BR, D_IN), lambda i: (i, 0)),
        pl.BlockSpec((D_IN, H1 * C1), lambda i: (0, 0)),
        pl.BlockSpec((H1, C1), lambda i: (0, 0)),
        pl.BlockSpec((H1, C1), lambda i: (0, 0)),
    ],
    out_specs=[
        pl.BlockSpec((BR, R1), lambda i: (i, 0)),
        pl.BlockSpec((BR, 8), lambda i: (i, 0)),
    ],
    out_shape=[
        jax.ShapeDtypeStruct((N, R1), jnp.float32),
        jax.ShapeDtypeStruct((N, 8), jnp.float32),
    ],
)

_mid_call = pl.pallas_call(
    _tc_mid_body,
    grid=(NB,),
    in_specs=[
        pl.BlockSpec((NC, BR, A1), lambda i: (0, i, 0)),
        pl.BlockSpec((1, D_IN), lambda i: (0, 0)),
        pl.BlockSpec((D_IN, H2 * C2), lambda i: (0, 0)),
        pl.BlockSpec((1, C2), lambda i: (0, 0)),
        pl.BlockSpec((1, C2), lambda i: (0, 0)),
    ],
    out_specs=[
        pl.BlockSpec((BR, R2), lambda i: (i, 0)),
        pl.BlockSpec((BR, 8), lambda i: (i, 0)),
    ],
    out_shape=[
        jax.ShapeDtypeStruct((N, R2), jnp.float32),
        jax.ShapeDtypeStruct((N, 8), jnp.float32),
    ],
)

_post_call = pl.pallas_call(
    _tc_post_body,
    grid=(NB,),
    in_specs=[
        pl.BlockSpec((NC, BR, A2), lambda i: (0, i, 0)),
        pl.BlockSpec((1, C2), lambda i: (0, 0)),
    ],
    out_specs=pl.BlockSpec((BR, C2), lambda i: (i, 0)),
    out_shape=jax.ShapeDtypeStruct((N, C2), jnp.float32),
)


def kernel(x, edge_index, W1, a_src1, a_dst1, b1, W2, a_src2, a_dst2, b2):
    esrc, edst = edge_index[0], edge_index[1]
    tab1, adst1 = _pre_call(x, W1, a_src1, a_dst1)
    accm1 = _l1_edges(tab1, adst1, esrc, edst)
    tab2, a2tab = _mid_call(accm1, b1.reshape(1, D_IN),
                            W2, a_src2.reshape(1, C2),
                            a_dst2.reshape(1, C2))
    accm2 = _l2_edges(tab2, a2tab, esrc, edst)
    return _post_call(accm2, b2.reshape(1, C2))


# speedup vs baseline: 74.3169x; 1.0096x over previous
import functools

import jax
import jax.numpy as jnp
from jax import lax
from jax.experimental import pallas as pl
from jax.experimental.pallas import tpu as pltpu
from jax.experimental.pallas import tpu_sc as plsc

N = 10000
E = 320000
D_IN = 128
H1, C1 = 8, 16
H2, C2 = 1, 64
HH = H1 // 2
CH = C2 // 2
R1 = 144
A1 = 144
D1 = 3
R2 = 80
A2 = 80

NC, NS = 2, 16
EPT = E // NS
CSZ = 80
NCH = EPT // CSZ
SB = 10
NSB = NCH // SB
DEPTH = 4
EPT2 = E // (NC * NS)
NCH2 = EPT2 // CSZ
SB2 = 5
NSB2 = NCH2 // SB2
NPH = 10112
ZR = NPH // NS

_mesh = plsc.VectorSubcoreMesh(core_axis_name="c", subcore_axis_name="s")
_sc_params = pltpu.CompilerParams(use_tc_tiling_on_sc=False,
                                  needs_layout_passes=False)


def _zero_fill(buf, rows, width):
    def _zrow(r, carry):
        for k in range(width // 16):
            buf[r, pl.ds(k * 16, 16)] = jnp.zeros((16,), jnp.float32)
        return carry
    lax.fori_loop(0, rows, _zrow, 0)


def _zero_acc_rows(acc, zbuf, zbase):
    for t in range(ZR // CSZ):
        pltpu.sync_copy(zbuf, acc.at[pl.ds(zbase + t * CSZ, CSZ)])
    rem = ZR - (ZR // CSZ) * CSZ
    if rem:
        pltpu.sync_copy(zbuf.at[pl.ds(0, rem)],
                        acc.at[pl.ds(zbase + ZR - rem, rem)])


@functools.partial(
    pl.kernel,
    out_type=jax.ShapeDtypeStruct((NC, NPH, A1), jnp.float32),
    mesh=_mesh,
    compiler_params=_sc_params,
    scratch_types=(
        [pltpu.VMEM((CSZ, R1), jnp.float32)] * D1
        + [pltpu.VMEM((CSZ, 8), jnp.float32)] * D1
        + [
            pltpu.VMEM((SB2 * CSZ,), jnp.int32),
            pltpu.VMEM((SB2 * CSZ,), jnp.int32),
            pltpu.VMEM_SHARED((NPH, A1), jnp.float32),
        ]
        + [pltpu.SemaphoreType.DMA] * (2 * D1)
    ),
)
def _l1_edges(tab_hbm, adst_hbm, esrc_hbm, edst_hbm, outm_hbm, *refs):
    bufi = refs[0:D1]
    dbuf = refs[D1:2 * D1]
    sidxb, didxb, acc_sh = refs[2 * D1:2 * D1 + 3]
    gsem = refs[2 * D1 + 3:2 * D1 + 3 + D1]
    dsem = refs[2 * D1 + 3 + D1:]
    cid = lax.axis_index("c")
    sid = lax.axis_index("s")
    ebase = (cid * NS + sid) * EPT2

    _zero_fill(bufi[0], CSZ, A1)
    _zero_acc_rows(acc_sh, bufi[0], sid * ZR)
    plsc.subcore_barrier()

    iota = lax.iota(jnp.int32, 16)

    def _super(s, carry):
        eb = ebase + s * (SB2 * CSZ)
        pltpu.sync_copy(esrc_hbm.at[pl.ds(eb, SB2 * CSZ)], sidxb)
        pltpu.sync_copy(edst_hbm.at[pl.ds(eb, SB2 * CSZ)], didxb)

        def _issue(k):
            sl = pl.ds(k * CSZ, CSZ)
            b = k % D1
            return (
                pltpu.async_copy(tab_hbm.at[sidxb.at[sl]], bufi[b], gsem[b]),
                pltpu.async_copy(adst_hbm.at[didxb.at[sl]], dbuf[b], dsem[b]),
            )

        hs = {k: _issue(k) for k in range(min(D1, SB2))}
        for k in range(SB2):
            ha, hb = hs.pop(k)
            ha.wait()
            hb.wait()
            b = k % D1
            for g in range(CSZ // 16):
                e16 = iota + g * 16
                for hd in range(H1):
                    hdv = jnp.full((16,), hd, jnp.int32)
                    a_s = plsc.load_gather(bufi[b], [e16, hdv])
                    a_d = plsc.load_gather(dbuf[b], [e16, hdv])
                    ew = a_s + a_d
                    ew = jnp.where(ew > 0.0, ew, 0.2 * ew)
                    w = jnp.exp(ew)
                    plsc.store_scatter(bufi[b], [e16, hdv], w)

            def _edge(q, carry2, b=b):
                for u in range(4):
                    i = q * 4 + u
                    wv = bufi[b][i, pl.ds(0, 16)]
                    for hd in range(H1):
                        ws = wv[hd]
                        sl = pl.ds(16 + hd * 16, 16)
                        bufi[b][i, sl] = bufi[b][i, sl] * ws
                return carry2
            lax.fori_loop(0, CSZ // 4, _edge, 0)
            pltpu.sync_copy(bufi[b],
                            acc_sh.at[didxb.at[pl.ds(k * CSZ, CSZ)]],
                            add=True)
            if k + D1 < SB2:
                hs[k + D1] = _issue(k + D1)
        return carry
    lax.fori_loop(0, NSB2, _super, 0)

    plsc.subcore_barrier()
    pltpu.sync_copy(acc_sh.at[pl.ds(sid * ZR, ZR)],
                    outm_hbm.at[cid, pl.ds(sid * ZR, ZR)])


@functools.partial(
    pl.kernel,
    out_type=jax.ShapeDtypeStruct((NC, NPH, A2), jnp.float32),
    mesh=_mesh,
    compiler_params=_sc_params,
    scratch_types=(
        [pltpu.VMEM((CSZ, R2), jnp.float32)] * DEPTH
        + [pltpu.VMEM((CSZ, 8), jnp.float32)] * DEPTH
        + [
            pltpu.VMEM((SB2 * CSZ,), jnp.int32),
            pltpu.VMEM((SB2 * CSZ,), jnp.int32),
            pltpu.VMEM_SHARED((NPH, A2), jnp.float32),
        ]
        + [pltpu.SemaphoreType.DMA] * (2 * DEPTH)
    ),
)
def _l2_edges(tab_hbm, a2_hbm, esrc_hbm, edst_hbm, outm_hbm, *refs):
    bufi = refs[0:DEPTH]
    dbuf = refs[DEPTH:2 * DEPTH]
    sidxb, didxb, accm_sh = refs[2 * DEPTH:2 * DEPTH + 3]
    gsem = refs[2 * DEPTH + 3:2 * DEPTH + 3 + DEPTH]
    dsem = refs[2 * DEPTH + 3 + DEPTH:]
    cid = lax.axis_index("c")
    sid = lax.axis_index("s")
    ebase = (cid * NS + sid) * EPT2

    _zero_fill(bufi[0], CSZ, A2)
    _zero_acc_rows(accm_sh, bufi[0], sid * ZR)
    plsc.subcore_barrier()

    iota = lax.iota(jnp.int32, 16)
    zero16 = jnp.zeros((16,), jnp.int32)

    def _super(s, carry):
        eb = ebase + s * (SB2 * CSZ)
        pltpu.sync_copy(esrc_hbm.at[pl.ds(eb, SB2 * CSZ)], sidxb)
        pltpu.sync_copy(edst_hbm.at[pl.ds(eb, SB2 * CSZ)], didxb)

        def _issue(k):
            sl = pl.ds(k * CSZ, CSZ)
            b = k % DEPTH
            return (
                pltpu.async_copy(tab_hbm.at[sidxb.at[sl]], bufi[b], gsem[b]),
                pltpu.async_copy(a2_hbm.at[didxb.at[sl]], dbuf[b], dsem[b]),
            )

        hs = {k: _issue(k) for k in range(min(DEPTH, SB2))}
        for k in range(SB2):
            ha, hb = hs.pop(k)
            ha.wait()
            hb.wait()
            b = k % DEPTH
            for g in range(CSZ // 16):
                e16 = iota + g * 16
                a_s = plsc.load_gather(bufi[b], [e16, zero16])
                a_d = plsc.load_gather(dbuf[b], [e16, zero16])
                ew = a_s + a_d
                ew = jnp.where(ew > 0.0, ew, 0.2 * ew)
                w = jnp.exp(ew)
                plsc.store_scatter(bufi[b], [e16, zero16], w)

            def _edge(q, carry2, b=b):
                for u in range(4):
                    i = q * 4 + u
                    ws = bufi[b][i, pl.ds(0, 16)][0]
                    for kk in range((A2 - 16) // 16):
                        sl = pl.ds(16 + kk * 16, 16)
                        bufi[b][i, sl] = bufi[b][i, sl] * ws
                return carry2
            lax.fori_loop(0, CSZ // 4, _edge, 0)
            pltpu.sync_copy(bufi[b],
                            accm_sh.at[didxb.at[pl.ds(k * CSZ, CSZ)]],
                            add=True)
            if k + DEPTH < SB2:
                hs[k + DEPTH] = _issue(k + DEPTH)
        return carry
    lax.fori_loop(0, NSB2, _super, 0)

    plsc.subcore_barrier()
    pltpu.sync_copy(accm_sh.at[pl.ds(sid * ZR, ZR)],
                    outm_hbm.at[cid, pl.ds(sid * ZR, ZR)])


BR = 1000
NB = N // BR


def _tc_pre_body(x_ref, w_ref, as_ref, ad_ref, tab_ref, adst_ref):
    h = jnp.dot(x_ref[...], w_ref[...], preferred_element_type=jnp.float32)
    hr = h.reshape(BR, H1, C1)
    asrc = (hr * as_ref[...][None]).sum(-1)
    adst = (hr * ad_ref[...][None]).sum(-1)
    z8 = jnp.zeros((BR, 8), jnp.float32)
    tab_ref[...] = jnp.concatenate([asrc, z8, h], axis=1)
    adst_ref[...] = adst


def _tc_mid_body(accm_ref, b1_ref, w2_ref, as2_ref, ad2_ref,
                 tab2_ref, a2tab_ref):
    wsum = accm_ref[0, :, 0:H1] + accm_ref[1, :, 0:H1]
    msg = accm_ref[0, :, 16:A1] + accm_ref[1, :, 16:A1]
    o = msg.reshape(BR, H1, C1) / (wsum[:, :, None] + 1e-16)
    o = (o + b1_ref[...].reshape(1, H1, C1)).reshape(BR, D_IN)
    o = jnp.where(o > 0.0, o, jnp.exp(o) - 1.0)
    h2 = jnp.dot(o, w2_ref[...], preferred_element_type=jnp.float32)
    asrc2 = (h2 * as2_ref[...]).sum(-1, keepdims=True)
    adst2 = (h2 * ad2_ref[...]).sum(-1, keepdims=True)
    z15 = jnp.zeros((BR, 15), jnp.float32)
    tab2_ref[...] = jnp.concatenate([asrc2, z15, h2], axis=1)
    a2tab_ref[...] = jnp.concatenate(
        [adst2, jnp.zeros((BR, 7), jnp.float32)], axis=1)


def _tc_post_body(accm_ref, b2_ref, out_ref):
    msg = accm_ref[0, :, 16:A2] + accm_ref[1, :, 16:A2]
    w = accm_ref[0, :, 0:1] + accm_ref[1, :, 0:1]
    out_ref[...] = msg / (w + 1e-16) + b2_ref[...]


_pre_call = pl.pallas_call(
    _tc_pre_body,
    grid=(NB,),
    in_specs=[
        pl.BlockSpec((BR, D_IN), lambda i: (i, 0)),
        pl.BlockSpec((D_IN, H1 * C1), lambda i: (0, 0)),
        pl.BlockSpec((H1, C1), lambda i: (0, 0)),
        pl.BlockSpec((H1, C1), lambda i: (0, 0)),
    ],
    out_specs=[
        pl.BlockSpec((BR, R1), lambda i: (i, 0)),
        pl.BlockSpec((BR, 8), lambda i: (i, 0)),
    ],
    out_shape=[
        jax.ShapeDtypeStruct((N, R1), jnp.float32),
        jax.ShapeDtypeStruct((N, 8), jnp.float32),
    ],
)

_mid_call = pl.pallas_call(
    _tc_mid_body,
    grid=(NB,),
    in_specs=[
        pl.BlockSpec((NC, BR, A1), lambda i: (0, i, 0)),
        pl.BlockSpec((1, D_IN), lambda i: (0, 0)),
        pl.BlockSpec((D_IN, H2 * C2), lambda i: (0, 0)),
        pl.BlockSpec((1, C2), lambda i: (0, 0)),
        pl.BlockSpec((1, C2), lambda i: (0, 0)),
    ],
    out_specs=[
        pl.BlockSpec((BR, R2), lambda i: (i, 0)),
        pl.BlockSpec((BR, 8), lambda i: (i, 0)),
    ],
    out_shape=[
        jax.ShapeDtypeStruct((N, R2), jnp.float32),
        jax.ShapeDtypeStruct((N, 8), jnp.float32),
    ],
)

_post_call = pl.pallas_call(
    _tc_post_body,
    grid=(NB,),
    in_specs=[
        pl.BlockSpec((NC, BR, A2), lambda i: (0, i, 0)),
        pl.BlockSpec((1, C2), lambda i: (0, 0)),
    ],
    out_specs=pl.BlockSpec((BR, C2), lambda i: (i, 0)),
    out_shape=jax.ShapeDtypeStruct((N, C2), jnp.float32),
)


def kernel(x, edge_index, W1, a_src1, a_dst1, b1, W2, a_src2, a_dst2, b2):
    esrc, edst = edge_index[0], edge_index[1]
    tab1, adst1 = _pre_call(x, W1, a_src1, a_dst1)
    accm1 = _l1_edges(tab1, adst1, esrc, edst)
    tab2, a2tab = _mid_call(accm1, b1.reshape(1, D_IN),
                            W2, a_src2.reshape(1, C2),
                            a_dst2.reshape(1, C2))
    accm2 = _l2_edges(tab2, a2tab, esrc, edst)
    return _post_call(accm2, b2.reshape(1, C2))
